# baseline (speedup 1.0000x reference)
import jax, jax.numpy as jnp
from jax.experimental import pallas as pl

_ATTN_DIM = 192


def _noop(x_ref, o_ref):
    o_ref[...] = x_ref[...]


def kernel(batch_users, batch_sequences, items_to_predict, edge_index, edge_type,
           node_no, short_edge_index, node_emb, rel_emb, W_long0, W_long1, W_short0,
           Wq, Wk, Wv, Wcq, Wck, Wcv, predict_w, predict_b):
    x = jnp.take(node_emb, node_no, axis=0)
    rate = jnp.ones((edge_type.shape[0], 1), dtype=x.dtype)
    src, dst = edge_index[0], edge_index[1]
    concat_states = []
    for W in (W_long0, W_long1):
        msg = x[src] * jnp.take(rel_emb, edge_type, axis=0) * rate
        agg = jax.ops.segment_sum(msg, dst, num_segments=x.shape[0])
        x = jax.nn.relu(agg @ W)
        concat_states.append(x)
    s_src, s_dst = short_edge_index[0], short_edge_index[1]
    for W in (W_short0,):
        msg = x[s_src]
        agg = jax.ops.segment_sum(msg, s_dst, num_segments=x.shape[0])
        x = jax.nn.relu(agg @ W)
        concat_states.append(x)
    concat = jnp.concatenate(concat_states, axis=1)
    user_embeddings = concat[batch_users]
    item_embeddings = concat[batch_sequences]
    q = item_embeddings @ Wq
    k = item_embeddings @ Wk
    v = item_embeddings @ Wv
    attn = jax.nn.softmax(jnp.einsum('bld,bmd->blm', q, k) / jnp.sqrt(float(_ATTN_DIM)), axis=-1)
    item_embeddings = item_embeddings + jnp.einsum('blm,bmd->bld', attn, v)
    qc = item_embeddings @ Wcq
    kc = user_embeddings @ Wck
    vc = user_embeddings @ Wcv
    score = jnp.einsum('bld,bd->bl', qc, kc) / jnp.sqrt(float(_ATTN_DIM))
    alpha = jax.nn.softmax(score, axis=-1)
    item_embeddings = item_embeddings + alpha[:, :, None] * vc[:, None, :]
    pe_w = jnp.take(predict_w, items_to_predict, axis=0)
    pe_b = jnp.take(predict_b, items_to_predict, axis=0)
    res = (pe_b + jnp.einsum('btd,bd->bt', pe_w, user_embeddings)[:, :, None])[..., 0]
    rel_score = jnp.einsum('bld,btd->blt', item_embeddings, pe_w).sum(axis=1)
    res = res + rel_score
    res = pl.pallas_call(_noop, out_shape=jax.ShapeDtypeStruct(res.shape, res.dtype))(res)
    return (res, user_embeddings, item_embeddings)


# speedup vs baseline: 2.0134x; 2.0134x over previous
import functools
import math

import jax
import jax.numpy as jnp
from jax import lax
from jax.experimental import pallas as pl
from jax.experimental.pallas import tpu as pltpu
from jax.experimental.pallas import tpu_sc as plsc

N = 50000
DIM = 64
Q = 16
NREL = 8
AD = 192
B = 1024
L = 50
T = 6
CHUNK = 1024
SUB = 128
ACC_ROWS = N + 48
E_LONG_PAD = 819200
E_SHORT_PAD = 212992
_NTILES = 16

_SC_PARAMS = pltpu.CompilerParams(use_tc_tiling_on_sc=False)


def _make_segsum(n_chunks_per_tile: int):
    mesh = plsc.VectorSubcoreMesh(core_axis_name="c", subcore_axis_name="s")
    scratch = [
        pltpu.VMEM((8, SUB), jnp.int32),
        pltpu.VMEM((8, SUB), jnp.int32),
        pltpu.VMEM((CHUNK, Q), jnp.float32),
        pltpu.VMEM_SHARED((ACC_ROWS, Q), jnp.float32),
        pltpu.SemaphoreType.DMA,
    ]
    out_type = jax.ShapeDtypeStruct((4 * N, Q), jnp.float32)

    @functools.partial(pl.kernel, mesh=mesh, out_type=out_type,
                       scratch_types=scratch, compiler_params=_SC_PARAMS)
    def seg_kernel(src4_hbm, dst2_hbm, x_hbm, out_hbm,
                   srcv, dstv, rows, acc, sem):
        cid = lax.axis_index("c")
        sid = lax.axis_index("s")

        zero16 = jnp.zeros((16,), jnp.float32)

        def zrow(i, carry):
            rows[i, pl.ds(0, 16)] = zero16
            return carry

        lax.fori_loop(0, CHUNK, zrow, 0)

        for p in range(2):
            qid = 2 * p + cid
            r0 = sid * 3128
            pltpu.sync_copy(rows.at[:, :], acc.at[pl.ds(r0, CHUNK), :])
            pltpu.sync_copy(rows.at[:, :], acc.at[pl.ds(r0 + CHUNK, CHUNK), :])
            pltpu.sync_copy(rows.at[:, :],
                            acc.at[pl.ds(r0 + 2 * CHUNK, CHUNK), :])
            pltpu.sync_copy(rows.at[pl.ds(0, 56), :],
                            acc.at[pl.ds(r0 + 3 * CHUNK, 56), :])
            plsc.subcore_barrier()

            def chunk_body(j, carry):
                c = sid + j * _NTILES
                row0 = c * 8
                pltpu.sync_copy(src4_hbm.at[qid, pl.ds(row0, 8), :], srcv)
                pltpu.sync_copy(dst2_hbm.at[pl.ds(row0, 8), :], dstv)
                for g in range(8):
                    pltpu.async_copy(x_hbm.at[srcv.at[g]],
                                     rows.at[pl.ds(g * SUB, SUB), :],
                                     sem).wait()
                for g in range(8):
                    pltpu.sync_copy(rows.at[pl.ds(g * SUB, SUB), :],
                                    acc.at[dstv.at[g]], add=True)
                return carry

            lax.fori_loop(0, n_chunks_per_tile, chunk_body, 0)
            plsc.subcore_barrier()
            o0 = sid * 3128

            @pl.when(sid < _NTILES - 1)
            def _full_stripe():
                pltpu.sync_copy(acc.at[pl.ds(o0, 3128), :],
                                out_hbm.at[pl.ds(qid * N + o0, 3128), :])

            @pl.when(sid == _NTILES - 1)
            def _last_stripe():
                pltpu.sync_copy(acc.at[pl.ds(o0, 3080), :],
                                out_hbm.at[pl.ds(qid * N + o0, 3080), :])

            if p == 0:
                plsc.subcore_barrier()
                lax.fori_loop(0, CHUNK, zrow, 0)

    return seg_kernel


_segsum_long = _make_segsum(E_LONG_PAD // CHUNK // _NTILES)
_segsum_short = _make_segsum(E_SHORT_PAD // CHUNK // _NTILES)

_BLK = 2000
_NRB = N // _BLK


def _scale_body(x_ref, rel_ref, out_ref):
    out_ref[...] = x_ref[...] * rel_ref[...].reshape(1, Q)


def _rel_scale(xq, rel4):
    grid = (4, _NRB, 8)
    return pl.pallas_call(
        _scale_body,
        grid=grid,
        in_specs=[
            pl.BlockSpec((_BLK, Q), lambda q, j, r: (q * _NRB + j, 0)),
            pl.BlockSpec((1, 1, Q), lambda q, j, r: (q * 8 + r, 0, 0)),
        ],
        out_specs=pl.BlockSpec(
            (_BLK, Q), lambda q, j, r: ((q * 8 + r) * _NRB + j, 0)),
        out_shape=jax.ShapeDtypeStruct((32 * N, Q), jnp.float32),
    )(xq, rel4.reshape(32, 1, Q))


def _mm_body(a0_ref, a1_ref, a2_ref, a3_ref, w_ref, out_ref):
    a = jnp.concatenate(
        [a0_ref[...], a1_ref[...], a2_ref[...], a3_ref[...]], axis=1)
    w = w_ref[...].reshape(DIM, Q)
    r = jnp.dot(a, w, preferred_element_type=jnp.float32)
    out_ref[...] = jnp.maximum(r, 0.0)


def _mm_relu(agg_q, w):
    grid = (4 * _NRB,)
    call = pl.pallas_call(
        _mm_body,
        grid=grid,
        in_specs=[
            pl.BlockSpec((_BLK, Q), lambda j: (0 * _NRB + j % _NRB, 0)),
            pl.BlockSpec((_BLK, Q), lambda j: (1 * _NRB + j % _NRB, 0)),
            pl.BlockSpec((_BLK, Q), lambda j: (2 * _NRB + j % _NRB, 0)),
            pl.BlockSpec((_BLK, Q), lambda j: (3 * _NRB + j % _NRB, 0)),
            pl.BlockSpec((1, DIM, Q), lambda j: (j // _NRB, 0, 0)),
        ],
        out_specs=pl.BlockSpec((_BLK, Q), lambda j: (j, 0)),
        out_shape=jax.ShapeDtypeStruct((4 * N, Q), jnp.float32),
    )
    w4 = jnp.stack([w[:, i * Q:(i + 1) * Q] for i in range(4)])
    return call(agg_q, agg_q, agg_q, agg_q, w4)


_IT_PER_W = B * L // 32
_IT_CHUNK = 400
_U_PER_W = B // 32
_P_PER_W = B * T // 32


def _gather_kernel_make():
    mesh = plsc.VectorSubcoreMesh(core_axis_name="c", subcore_axis_name="s")
    scratch = [
        pltpu.VMEM((_IT_PER_W,), jnp.int32),
        pltpu.VMEM((_IT_CHUNK, Q), jnp.float32),
        pltpu.VMEM((_P_PER_W, AD), jnp.float32),
        pltpu.VMEM((_P_PER_W,), jnp.float32),
        pltpu.SemaphoreType.DMA,
    ]
    out_type = [
        jax.ShapeDtypeStruct((B, AD), jnp.float32),
        jax.ShapeDtypeStruct((B * L, AD), jnp.float32),
        jax.ShapeDtypeStruct((B * T, AD), jnp.float32),
        jax.ShapeDtypeStruct((B * T,), jnp.float32),
    ]

    @functools.partial(pl.kernel, mesh=mesh, out_type=out_type,
                       scratch_types=scratch, compiler_params=_SC_PARAMS)
    def gather_kernel(x1, x2, x3, bu4, seq4, itp, pw, pb,
                      out_u, out_i, out_w, out_b,
                      idxv, rows, prow, pbv, sem):
        cid = lax.axis_index("c")
        sid = lax.axis_index("s")
        w = sid * 2 + cid

        ib = w * _IT_PER_W
        subs = ((0, 128), (128, 128), (256, 128), (384, 16))
        for q in range(4):
            pltpu.sync_copy(seq4.at[q, pl.ds(ib, _IT_PER_W)], idxv)
            for ch in range(4):
                for t, tb in enumerate((x1, x2, x3)):
                    for off, sz in subs:
                        pltpu.async_copy(
                            tb.at[idxv.at[pl.ds(ch * _IT_CHUNK + off, sz)]],
                            rows.at[pl.ds(off, sz), :], sem).wait()
                    pltpu.sync_copy(
                        rows,
                        out_i.at[pl.ds(ib + ch * _IT_CHUNK, _IT_CHUNK),
                                 pl.ds(t * DIM + q * Q, Q)])

        ub = w * _U_PER_W
        for q in range(4):
            pltpu.sync_copy(bu4.at[q, pl.ds(ub, _U_PER_W)],
                            idxv.at[pl.ds(0, _U_PER_W)])
            for t, tb in enumerate((x1, x2, x3)):
                pltpu.async_copy(tb.at[idxv.at[pl.ds(0, _U_PER_W)]],
                                 rows.at[pl.ds(0, _U_PER_W), :], sem).wait()
                pltpu.sync_copy(
                    rows.at[pl.ds(0, _U_PER_W), :],
                    out_u.at[pl.ds(ub, _U_PER_W),
                             pl.ds(t * DIM + q * Q, Q)])

        pbase = w * _P_PER_W
        pltpu.sync_copy(itp.at[pl.ds(pbase, _P_PER_W)],
                        idxv.at[pl.ds(0, _P_PER_W)])
        for off, sz in ((0, 128), (128, 64)):
            pltpu.async_copy(pw.at[idxv.at[pl.ds(off, sz)]],
                             prow.at[pl.ds(off, sz), :], sem).wait()
            pltpu.async_copy(pb.at[idxv.at[pl.ds(off, sz)]],
                             pbv.at[pl.ds(off, sz)], sem).wait()
        pltpu.sync_copy(prow, out_w.at[pl.ds(pbase, _P_PER_W), :])
        pltpu.sync_copy(pbv, out_b.at[pl.ds(pbase, _P_PER_W)])

    return gather_kernel


_gather_rows = _gather_kernel_make()

_NB = 32
_SCALE = 1.0 / math.sqrt(float(AD))


def _attn_body(user_ref, item_ref, pew_ref, peb_ref,
               wq_ref, wk_ref, wv_ref, wcq_ref, wck_ref, wcv_ref,
               res_ref, item_out_ref):
    item = item_ref[...]
    user = user_ref[...]
    q = jnp.dot(item, wq_ref[...], preferred_element_type=jnp.float32)
    k = jnp.dot(item, wk_ref[...], preferred_element_type=jnp.float32)
    v = jnp.dot(item, wv_ref[...], preferred_element_type=jnp.float32)
    q3 = q.reshape(_NB, L, AD)
    k3 = k.reshape(_NB, L, AD)
    v3 = v.reshape(_NB, L, AD)
    s = lax.dot_general(q3, k3, (((2,), (2,)), ((0,), (0,))),
                        preferred_element_type=jnp.float32) * _SCALE
    s = s - jnp.max(s, axis=-1, keepdims=True)
    e = jnp.exp(s)
    attn = e / jnp.sum(e, axis=-1, keepdims=True)
    it3 = item.reshape(_NB, L, AD) + lax.dot_general(
        attn, v3, (((2,), (1,)), ((0,), (0,))),
        preferred_element_type=jnp.float32)
    qc = jnp.dot(it3.reshape(_NB * L, AD), wcq_ref[...],
                 preferred_element_type=jnp.float32).reshape(_NB, L, AD)
    kc = jnp.dot(user, wck_ref[...], preferred_element_type=jnp.float32)
    vc = jnp.dot(user, wcv_ref[...], preferred_element_type=jnp.float32)
    score = jnp.sum(qc * kc[:, None, :], axis=-1) * _SCALE
    score = score - jnp.max(score, axis=-1, keepdims=True)
    es = jnp.exp(score)
    alpha = es / jnp.sum(es, axis=-1, keepdims=True)
    it4 = it3 + alpha[:, :, None] * vc[:, None, :]
    item_out_ref[...] = it4.reshape(_NB * L, AD)
    isum = jnp.sum(it4, axis=1)
    up = user + isum
    pew = pew_ref[...].reshape(_NB, T, AD)
    res_ref[...] = peb_ref[...] + jnp.sum(pew * up[:, None, :], axis=-1)


def _attention(user0, item0, pew, peb, wq, wk, wv, wcq, wck, wcv):
    grid = (B // _NB,)
    wspec = pl.BlockSpec((AD, AD), lambda i: (0, 0))
    return pl.pallas_call(
        _attn_body,
        grid=grid,
        in_specs=[
            pl.BlockSpec((_NB, AD), lambda i: (i, 0)),
            pl.BlockSpec((_NB * L, AD), lambda i: (i, 0)),
            pl.BlockSpec((_NB * T, AD), lambda i: (i, 0)),
            pl.BlockSpec((_NB, T), lambda i: (i, 0)),
            wspec, wspec, wspec, wspec, wspec, wspec,
        ],
        out_specs=[
            pl.BlockSpec((_NB, T), lambda i: (i, 0)),
            pl.BlockSpec((_NB * L, AD), lambda i: (i, 0)),
        ],
        out_shape=[
            jax.ShapeDtypeStruct((B, T), jnp.float32),
            jax.ShapeDtypeStruct((B * L, AD), jnp.float32),
        ],
    )(user0, item0, pew, peb, wq, wk, wv, wcq, wck, wcv)


def _pad_edges(gidx, dst, e_pad, sect_rows):
    e = gidx.shape[0]
    npad = e_pad - e
    padi = jnp.arange(npad, dtype=jnp.int32)
    g_p = jnp.concatenate([gidx, padi % N])
    dst_p = jnp.concatenate([dst, N + (padi % (ACC_ROWS - N))])
    g4 = jnp.stack([g_p + q * sect_rows for q in range(4)])
    g4 = g4.reshape(4, e_pad // SUB, SUB)
    dst2 = dst_p.reshape(e_pad // SUB, SUB)
    return g4, dst2


def _to_quarters(x):
    m = x.shape[0]
    return x.reshape(m, 4, Q).transpose(1, 0, 2).reshape(4 * m, Q)


def kernel(batch_users, batch_sequences, items_to_predict, edge_index,
           edge_type, node_no, short_edge_index, node_emb, rel_emb,
           W_long0, W_long1, W_short0, Wq, Wk, Wv, Wcq, Wck, Wcv,
           predict_w, predict_b):
    src, dst = edge_index[0], edge_index[1]
    s_src, s_dst = short_edge_index[0], short_edge_index[1]
    gidxL = edge_type * N + src
    g4L, d2L = _pad_edges(gidxL, dst, E_LONG_PAD, 8 * N)
    g4S, d2S = _pad_edges(s_src, s_dst, E_SHORT_PAD, N)

    xq0 = _to_quarters(node_emb)
    rel4 = _to_quarters(rel_emb)

    sc0 = _rel_scale(xq0, rel4)
    agg1 = _segsum_long(g4L, d2L, sc0)
    x1q = _mm_relu(agg1, W_long0)
    sc1 = _rel_scale(x1q, rel4)
    agg2 = _segsum_long(g4L, d2L, sc1)
    x2q = _mm_relu(agg2, W_long1)
    agg3 = _segsum_short(g4S, d2S, x2q)
    x3q = _mm_relu(agg3, W_short0)

    seq = batch_sequences.reshape(-1)
    seq4 = jnp.stack([seq + q * N for q in range(4)])
    bu4 = jnp.stack([batch_users + q * N for q in range(4)])
    user0, item0, pew, peb = _gather_rows(
        x1q, x2q, x3q, bu4, seq4,
        items_to_predict.reshape(-1), predict_w, predict_b[:, 0])

    res, item_out = _attention(user0, item0, pew, peb.reshape(B, T),
                               Wq, Wk, Wv, Wcq, Wck, Wcv)
    return (res, user0, item_out.reshape(B, L, AD))


# speedup vs baseline: 3.6392x; 1.8075x over previous
import functools
import math

import jax
import jax.numpy as jnp
from jax import lax
from jax.experimental import pallas as pl
from jax.experimental.pallas import tpu as pltpu
from jax.experimental.pallas import tpu_sc as plsc

N = 50000
DIM = 64
Q = 16
NREL = 8
AD = 192
B = 1024
L = 50
T = 6
CHUNK = 1024
SUB = 128
ACC_ROWS = N + 48
E_LONG_PAD = 819200
E_SHORT_PAD = 212992
_NTILES = 16

_SC_PARAMS = pltpu.CompilerParams(use_tc_tiling_on_sc=False)


def _make_segsum(n_chunks_per_tile: int):
    mesh = plsc.VectorSubcoreMesh(core_axis_name="c", subcore_axis_name="s")
    scratch = [
        pltpu.VMEM((CHUNK,), jnp.int32),
        pltpu.VMEM((CHUNK,), jnp.int32),
        pltpu.VMEM((CHUNK, Q), jnp.float32),
        pltpu.VMEM_SHARED((ACC_ROWS, Q), jnp.float32),
        pltpu.SemaphoreType.DMA,
    ]
    out_type = jax.ShapeDtypeStruct((4 * N, Q), jnp.float32)

    @functools.partial(pl.kernel, mesh=mesh, out_type=out_type,
                       scratch_types=scratch, compiler_params=_SC_PARAMS)
    def seg_kernel(src4_hbm, dst2_hbm, x_hbm, out_hbm,
                   srcv, dstv, rows, acc, sem):
        cid = lax.axis_index("c")
        sid = lax.axis_index("s")

        zero16 = jnp.zeros((16,), jnp.float32)

        def zrow(i, carry):
            rows[i, pl.ds(0, 16)] = zero16
            return carry

        lax.fori_loop(0, CHUNK, zrow, 0, unroll=8)

        for p in range(2):
            qid = 2 * p + cid
            r0 = sid * 3128
            pltpu.sync_copy(rows.at[:, :], acc.at[pl.ds(r0, CHUNK), :])
            pltpu.sync_copy(rows.at[:, :], acc.at[pl.ds(r0 + CHUNK, CHUNK), :])
            pltpu.sync_copy(rows.at[:, :],
                            acc.at[pl.ds(r0 + 2 * CHUNK, CHUNK), :])
            pltpu.sync_copy(rows.at[pl.ds(0, 56), :],
                            acc.at[pl.ds(r0 + 3 * CHUNK, 56), :])
            plsc.subcore_barrier()

            def chunk_body(j, carry):
                c = sid + j * _NTILES
                e0 = c * CHUNK
                pltpu.sync_copy(src4_hbm.at[qid, pl.ds(e0, CHUNK)], srcv)
                pltpu.sync_copy(dst2_hbm.at[pl.ds(e0, CHUNK)], dstv)
                pltpu.async_copy(x_hbm.at[srcv], rows, sem).wait()
                pltpu.sync_copy(rows, acc.at[dstv], add=True)
                return carry

            lax.fori_loop(0, n_chunks_per_tile, chunk_body, 0)
            plsc.subcore_barrier()
            o0 = sid * 3128

            @pl.when(sid < _NTILES - 1)
            def _full_stripe():
                pltpu.sync_copy(acc.at[pl.ds(o0, 3128), :],
                                out_hbm.at[pl.ds(qid * N + o0, 3128), :])

            @pl.when(sid == _NTILES - 1)
            def _last_stripe():
                pltpu.sync_copy(acc.at[pl.ds(o0, 3080), :],
                                out_hbm.at[pl.ds(qid * N + o0, 3080), :])

            if p == 0:
                plsc.subcore_barrier()
                lax.fori_loop(0, CHUNK, zrow, 0)

    return seg_kernel


_segsum_long = _make_segsum(E_LONG_PAD // CHUNK // _NTILES)
_segsum_short = _make_segsum(E_SHORT_PAD // CHUNK // _NTILES)

_BLK = 2000
_NRB = N // _BLK
_RS_CHUNK = 625
_RS_PER_W = N // 8


def _rel_scale_make():
    mesh = plsc.VectorSubcoreMesh(core_axis_name="c", subcore_axis_name="s")
    scratch = [
        pltpu.VMEM((_RS_CHUNK, Q), jnp.float32),
        pltpu.VMEM((_RS_CHUNK, Q), jnp.float32),
        pltpu.VMEM((NREL, Q), jnp.float32),
        pltpu.SemaphoreType.DMA,
    ]
    out_type = jax.ShapeDtypeStruct((32 * N, Q), jnp.float32)

    @functools.partial(pl.kernel, mesh=mesh, out_type=out_type,
                       scratch_types=scratch, compiler_params=_SC_PARAMS)
    def rs_kernel(x_hbm, rel_hbm, out_hbm, xbuf, obuf, relv, sem):
        cid = lax.axis_index("c")
        sid = lax.axis_index("s")
        w = sid * 2 + cid
        qid = w // 8
        si = w % 8
        base = qid * N + si * _RS_PER_W
        pltpu.sync_copy(rel_hbm.at[pl.ds(qid * NREL, NREL), :], relv)
        for ch in range(_RS_PER_W // _RS_CHUNK):
            pltpu.sync_copy(
                x_hbm.at[pl.ds(base + ch * _RS_CHUNK, _RS_CHUNK), :], xbuf)
            for r in range(NREL):
                rv = relv[r, pl.ds(0, Q)]

                def mul(i, carry, rv=rv):
                    obuf[i, pl.ds(0, Q)] = xbuf[i, pl.ds(0, Q)] * rv
                    return carry

                lax.fori_loop(0, _RS_CHUNK, mul, 0, unroll=8)
                o0 = ((qid * NREL + r) * N + si * _RS_PER_W
                      + ch * _RS_CHUNK)
                pltpu.sync_copy(obuf, out_hbm.at[pl.ds(o0, _RS_CHUNK), :])

    return rs_kernel


_rel_scale = _rel_scale_make()


def _mm_body(a0_ref, a1_ref, a2_ref, a3_ref, w_ref, out_ref):
    a = jnp.concatenate(
        [a0_ref[...], a1_ref[...], a2_ref[...], a3_ref[...]], axis=1)
    w = w_ref[...].reshape(DIM, Q)
    r = jnp.dot(a, w, preferred_element_type=jnp.float32)
    out_ref[...] = jnp.maximum(r, 0.0)


def _mm_relu(agg_q, w):
    grid = (4 * _NRB,)
    call = pl.pallas_call(
        _mm_body,
        grid=grid,
        in_specs=[
            pl.BlockSpec((_BLK, Q), lambda j: (0 * _NRB + j % _NRB, 0)),
            pl.BlockSpec((_BLK, Q), lambda j: (1 * _NRB + j % _NRB, 0)),
            pl.BlockSpec((_BLK, Q), lambda j: (2 * _NRB + j % _NRB, 0)),
            pl.BlockSpec((_BLK, Q), lambda j: (3 * _NRB + j % _NRB, 0)),
            pl.BlockSpec((1, DIM, Q), lambda j: (j // _NRB, 0, 0)),
        ],
        out_specs=pl.BlockSpec((_BLK, Q), lambda j: (j, 0)),
        out_shape=jax.ShapeDtypeStruct((4 * N, Q), jnp.float32),
    )
    w4 = jnp.stack([w[:, i * Q:(i + 1) * Q] for i in range(4)])
    return call(agg_q, agg_q, agg_q, agg_q, w4)


_IT_PER_W = B * L // 32
_IT_CHUNK = 400
_U_PER_W = B // 32
_P_PER_W = B * T // 32


def _gather_kernel_make():
    mesh = plsc.VectorSubcoreMesh(core_axis_name="c", subcore_axis_name="s")
    scratch = [
        pltpu.VMEM((_IT_PER_W,), jnp.int32),
        pltpu.VMEM((_IT_CHUNK, Q), jnp.float32),
        pltpu.VMEM((_P_PER_W, AD), jnp.float32),
        pltpu.VMEM((_P_PER_W,), jnp.float32),
        pltpu.SemaphoreType.DMA,
    ]
    out_type = [
        jax.ShapeDtypeStruct((B, AD), jnp.float32),
        jax.ShapeDtypeStruct((B * L, AD), jnp.float32),
        jax.ShapeDtypeStruct((B * T, AD), jnp.float32),
        jax.ShapeDtypeStruct((B * T,), jnp.float32),
    ]

    @functools.partial(pl.kernel, mesh=mesh, out_type=out_type,
                       scratch_types=scratch, compiler_params=_SC_PARAMS)
    def gather_kernel(x1, x2, x3, bu4, seq4, itp, pw, pb,
                      out_u, out_i, out_w, out_b,
                      idxv, rows, prow, pbv, sem):
        cid = lax.axis_index("c")
        sid = lax.axis_index("s")
        w = sid * 2 + cid

        ib = w * _IT_PER_W
        subs = ((0, 128), (128, 128), (256, 128), (384, 16))
        for q in range(4):
            pltpu.sync_copy(seq4.at[q, pl.ds(ib, _IT_PER_W)], idxv)
            for ch in range(4):
                for t, tb in enumerate((x1, x2, x3)):
                    for off, sz in subs:
                        pltpu.async_copy(
                            tb.at[idxv.at[pl.ds(ch * _IT_CHUNK + off, sz)]],
                            rows.at[pl.ds(off, sz), :], sem).wait()
                    pltpu.sync_copy(
                        rows,
                        out_i.at[pl.ds(ib + ch * _IT_CHUNK, _IT_CHUNK),
                                 pl.ds(t * DIM + q * Q, Q)])

        ub = w * _U_PER_W
        for q in range(4):
            pltpu.sync_copy(bu4.at[q, pl.ds(ub, _U_PER_W)],
                            idxv.at[pl.ds(0, _U_PER_W)])
            for t, tb in enumerate((x1, x2, x3)):
                pltpu.async_copy(tb.at[idxv.at[pl.ds(0, _U_PER_W)]],
                                 rows.at[pl.ds(0, _U_PER_W), :], sem).wait()
                pltpu.sync_copy(
                    rows.at[pl.ds(0, _U_PER_W), :],
                    out_u.at[pl.ds(ub, _U_PER_W),
                             pl.ds(t * DIM + q * Q, Q)])

        pbase = w * _P_PER_W
        pltpu.sync_copy(itp.at[pl.ds(pbase, _P_PER_W)],
                        idxv.at[pl.ds(0, _P_PER_W)])
        for off, sz in ((0, 128), (128, 64)):
            pltpu.async_copy(pw.at[idxv.at[pl.ds(off, sz)]],
                             prow.at[pl.ds(off, sz), :], sem).wait()
            pltpu.async_copy(pb.at[idxv.at[pl.ds(off, sz)]],
                             pbv.at[pl.ds(off, sz)], sem).wait()
        pltpu.sync_copy(prow, out_w.at[pl.ds(pbase, _P_PER_W), :])
        pltpu.sync_copy(pbv, out_b.at[pl.ds(pbase, _P_PER_W)])

    return gather_kernel


_gather_rows = _gather_kernel_make()

_NB = 32
_SCALE = 1.0 / math.sqrt(float(AD))


def _attn_body(user_ref, item_ref, pew_ref, peb_ref,
               wq_ref, wk_ref, wv_ref, wcq_ref, wck_ref, wcv_ref,
               res_ref, item_out_ref):
    item = item_ref[...]
    user = user_ref[...]
    q = jnp.dot(item, wq_ref[...], preferred_element_type=jnp.float32)
    k = jnp.dot(item, wk_ref[...], preferred_element_type=jnp.float32)
    v = jnp.dot(item, wv_ref[...], preferred_element_type=jnp.float32)
    q3 = q.reshape(_NB, L, AD)
    k3 = k.reshape(_NB, L, AD)
    v3 = v.reshape(_NB, L, AD)
    s = lax.dot_general(q3, k3, (((2,), (2,)), ((0,), (0,))),
                        preferred_element_type=jnp.float32) * _SCALE
    s = s - jnp.max(s, axis=-1, keepdims=True)
    e = jnp.exp(s)
    attn = e / jnp.sum(e, axis=-1, keepdims=True)
    it3 = item.reshape(_NB, L, AD) + lax.dot_general(
        attn, v3, (((2,), (1,)), ((0,), (0,))),
        preferred_element_type=jnp.float32)
    qc = jnp.dot(it3.reshape(_NB * L, AD), wcq_ref[...],
                 preferred_element_type=jnp.float32).reshape(_NB, L, AD)
    kc = jnp.dot(user, wck_ref[...], preferred_element_type=jnp.float32)
    vc = jnp.dot(user, wcv_ref[...], preferred_element_type=jnp.float32)
    score = jnp.sum(qc * kc[:, None, :], axis=-1) * _SCALE
    score = score - jnp.max(score, axis=-1, keepdims=True)
    es = jnp.exp(score)
    alpha = es / jnp.sum(es, axis=-1, keepdims=True)
    it4 = it3 + alpha[:, :, None] * vc[:, None, :]
    item_out_ref[...] = it4.reshape(_NB * L, AD)
    isum = jnp.sum(it4, axis=1)
    up = user + isum
    pew = pew_ref[...].reshape(_NB, T, AD)
    res_ref[...] = peb_ref[...] + jnp.sum(pew * up[:, None, :], axis=-1)


def _attention(user0, item0, pew, peb, wq, wk, wv, wcq, wck, wcv):
    grid = (B // _NB,)
    wspec = pl.BlockSpec((AD, AD), lambda i: (0, 0))
    return pl.pallas_call(
        _attn_body,
        grid=grid,
        in_specs=[
            pl.BlockSpec((_NB, AD), lambda i: (i, 0)),
            pl.BlockSpec((_NB * L, AD), lambda i: (i, 0)),
            pl.BlockSpec((_NB * T, AD), lambda i: (i, 0)),
            pl.BlockSpec((_NB, T), lambda i: (i, 0)),
            wspec, wspec, wspec, wspec, wspec, wspec,
        ],
        out_specs=[
            pl.BlockSpec((_NB, T), lambda i: (i, 0)),
            pl.BlockSpec((_NB * L, AD), lambda i: (i, 0)),
        ],
        out_shape=[
            jax.ShapeDtypeStruct((B, T), jnp.float32),
            jax.ShapeDtypeStruct((B * L, AD), jnp.float32),
        ],
    )(user0, item0, pew, peb, wq, wk, wv, wcq, wck, wcv)


def _pad_edges(gidx, dst, e_pad, sect_rows):
    e = gidx.shape[0]
    npad = e_pad - e
    padi = jnp.arange(npad, dtype=jnp.int32)
    g_p = jnp.concatenate([gidx, padi % N])
    dst_p = jnp.concatenate([dst, N + (padi % (ACC_ROWS - N))])
    g4 = jnp.stack([g_p + q * sect_rows for q in range(4)])
    return g4, dst_p


def _to_quarters(x):
    m = x.shape[0]
    return x.reshape(m, 4, Q).transpose(1, 0, 2).reshape(4 * m, Q)


def kernel(batch_users, batch_sequences, items_to_predict, edge_index,
           edge_type, node_no, short_edge_index, node_emb, rel_emb,
           W_long0, W_long1, W_short0, Wq, Wk, Wv, Wcq, Wck, Wcv,
           predict_w, predict_b):
    src, dst = edge_index[0], edge_index[1]
    s_src, s_dst = short_edge_index[0], short_edge_index[1]
    gidxL = edge_type * N + src
    g4L, d2L = _pad_edges(gidxL, dst, E_LONG_PAD, 8 * N)
    g4S, d2S = _pad_edges(s_src, s_dst, E_SHORT_PAD, N)

    xq0 = _to_quarters(node_emb)
    rel4 = _to_quarters(rel_emb)

    sc0 = _rel_scale(xq0, rel4)
    agg1 = _segsum_long(g4L, d2L, sc0)
    x1q = _mm_relu(agg1, W_long0)
    sc1 = _rel_scale(x1q, rel4)
    agg2 = _segsum_long(g4L, d2L, sc1)
    x2q = _mm_relu(agg2, W_long1)
    agg3 = _segsum_short(g4S, d2S, x2q)
    x3q = _mm_relu(agg3, W_short0)

    seq = batch_sequences.reshape(-1)
    seq4 = jnp.stack([seq + q * N for q in range(4)])
    bu4 = jnp.stack([batch_users + q * N for q in range(4)])
    user0, item0, pew, peb = _gather_rows(
        x1q, x2q, x3q, bu4, seq4,
        items_to_predict.reshape(-1), predict_w, predict_b[:, 0])

    res, item_out = _attention(user0, item0, pew, peb.reshape(B, T),
                               Wq, Wk, Wv, Wcq, Wck, Wcv)
    return (res, user0, item_out.reshape(B, L, AD))


# speedup vs baseline: 4.2501x; 1.1679x over previous
import functools
import math

import jax
import jax.numpy as jnp
from jax import lax
from jax.experimental import pallas as pl
from jax.experimental.pallas import tpu as pltpu
from jax.experimental.pallas import tpu_sc as plsc

N = 50000
DIM = 64
Q = 16
NREL = 8
AD = 192
B = 1024
L = 50
T = 6
CHUNK = 1024
SUB = 128
ACC_ROWS = N + 48
E_LONG_PAD = 819200
E_SHORT_PAD = 229376
_NTILES = 16

_SC_PARAMS = pltpu.CompilerParams(use_tc_tiling_on_sc=False)


def _make_segsum(n_chunks_per_tile: int):
    mesh = plsc.VectorSubcoreMesh(core_axis_name="c", subcore_axis_name="s")
    scratch = [
        pltpu.VMEM((CHUNK,), jnp.int32),
        pltpu.VMEM((CHUNK,), jnp.int32),
        pltpu.VMEM((CHUNK,), jnp.int32),
        pltpu.VMEM((CHUNK,), jnp.int32),
        pltpu.VMEM((CHUNK, Q), jnp.float32),
        pltpu.VMEM((CHUNK, Q), jnp.float32),
        pltpu.VMEM_SHARED((ACC_ROWS, Q), jnp.float32),
        pltpu.SemaphoreType.DMA,
        pltpu.SemaphoreType.DMA,
    ]
    out_type = jax.ShapeDtypeStruct((4 * N, Q), jnp.float32)
    assert n_chunks_per_tile % 2 == 0

    @functools.partial(pl.kernel, mesh=mesh, out_type=out_type,
                       scratch_types=scratch, compiler_params=_SC_PARAMS)
    def seg_kernel(src4_hbm, dst2_hbm, x_hbm, out_hbm,
                   srcv0, srcv1, dstv0, dstv1, rows0, rows1, acc,
                   sem0, sem1):
        cid = lax.axis_index("c")
        sid = lax.axis_index("s")

        zero16 = jnp.zeros((16,), jnp.float32)

        def zrow0(i, carry):
            rows0[i, pl.ds(0, 16)] = zero16
            return carry

        def chunk_off(j):
            return (sid + j * _NTILES) * CHUNK

        for p in range(2):
            qid = 2 * p + cid
            lax.fori_loop(0, CHUNK, zrow0, 0, unroll=8)
            r0 = sid * 3128
            pltpu.sync_copy(rows0.at[:, :], acc.at[pl.ds(r0, CHUNK), :])
            pltpu.sync_copy(rows0.at[:, :],
                            acc.at[pl.ds(r0 + CHUNK, CHUNK), :])
            pltpu.sync_copy(rows0.at[:, :],
                            acc.at[pl.ds(r0 + 2 * CHUNK, CHUNK), :])
            pltpu.sync_copy(rows0.at[pl.ds(0, 56), :],
                            acc.at[pl.ds(r0 + 3 * CHUNK, 56), :])
            plsc.subcore_barrier()

            nhalf = n_chunks_per_tile // 2
            e0 = chunk_off(0)
            pltpu.sync_copy(src4_hbm.at[qid, pl.ds(e0, CHUNK)], srcv0)
            pltpu.sync_copy(dst2_hbm.at[pl.ds(e0, CHUNK)], dstv0)
            pltpu.async_copy(x_hbm.at[srcv0], rows0, sem0)

            def pair_body(k, carry):
                e1 = chunk_off(2 * k + 1)
                pltpu.sync_copy(src4_hbm.at[qid, pl.ds(e1, CHUNK)], srcv1)
                pltpu.sync_copy(dst2_hbm.at[pl.ds(e1, CHUNK)], dstv1)
                pltpu.async_copy(x_hbm.at[srcv1], rows1, sem1)
                pltpu.make_async_copy(x_hbm.at[srcv0], rows0, sem0).wait()
                pltpu.sync_copy(rows0, acc.at[dstv0], add=True)

                @pl.when(k < nhalf - 1)
                def _prefetch_even():
                    e2 = chunk_off(2 * k + 2)
                    pltpu.sync_copy(src4_hbm.at[qid, pl.ds(e2, CHUNK)],
                                    srcv0)
                    pltpu.sync_copy(dst2_hbm.at[pl.ds(e2, CHUNK)], dstv0)
                    pltpu.async_copy(x_hbm.at[srcv0], rows0, sem0)

                pltpu.make_async_copy(x_hbm.at[srcv1], rows1, sem1).wait()
                pltpu.sync_copy(rows1, acc.at[dstv1], add=True)
                return carry

            lax.fori_loop(0, nhalf, pair_body, 0)
            plsc.subcore_barrier()
            o0 = sid * 3128

            @pl.when(sid < _NTILES - 1)
            def _full_stripe():
                pltpu.sync_copy(acc.at[pl.ds(o0, 3128), :],
                                out_hbm.at[pl.ds(qid * N + o0, 3128), :])

            @pl.when(sid == _NTILES - 1)
            def _last_stripe():
                pltpu.sync_copy(acc.at[pl.ds(o0, 3080), :],
                                out_hbm.at[pl.ds(qid * N + o0, 3080), :])

    return seg_kernel


_segsum_long = _make_segsum(E_LONG_PAD // CHUNK // _NTILES)
_segsum_short = _make_segsum(E_SHORT_PAD // CHUNK // _NTILES)

_BLK = 2000
_NRB = N // _BLK
_RS_CHUNK = 625
_RS_PER_W = N // 8


def _rel_scale_make():
    mesh = plsc.VectorSubcoreMesh(core_axis_name="c", subcore_axis_name="s")
    scratch = [
        pltpu.VMEM((_RS_CHUNK, Q), jnp.float32),
        pltpu.VMEM((_RS_CHUNK, Q), jnp.float32),
        pltpu.VMEM((_RS_CHUNK, Q), jnp.float32),
        pltpu.VMEM((NREL, Q), jnp.float32),
        pltpu.SemaphoreType.DMA,
        pltpu.SemaphoreType.DMA,
    ]
    out_type = jax.ShapeDtypeStruct((32 * N, Q), jnp.float32)

    @functools.partial(pl.kernel, mesh=mesh, out_type=out_type,
                       scratch_types=scratch, compiler_params=_SC_PARAMS)
    def rs_kernel(x_hbm, rel_hbm, out_hbm, xbuf, obuf0, obuf1, relv,
                  sem0, sem1):
        cid = lax.axis_index("c")
        sid = lax.axis_index("s")
        w = sid * 2 + cid
        qid = w // 8
        si = w % 8
        base = qid * N + si * _RS_PER_W
        pltpu.sync_copy(rel_hbm.at[pl.ds(qid * NREL, NREL), :], relv)
        obufs = (obuf0, obuf1)
        sems = (sem0, sem1)
        pend = [None, None]
        for ch in range(_RS_PER_W // _RS_CHUNK):
            pltpu.sync_copy(
                x_hbm.at[pl.ds(base + ch * _RS_CHUNK, _RS_CHUNK), :], xbuf)
            for r in range(NREL):
                b = r % 2
                if pend[b] is not None:
                    pend[b].wait()
                ob, rv = obufs[b], relv[r, pl.ds(0, Q)]

                def mul(i, carry, ob=ob, rv=rv):
                    ob[i, pl.ds(0, Q)] = xbuf[i, pl.ds(0, Q)] * rv
                    return carry

                lax.fori_loop(0, _RS_CHUNK, mul, 0, unroll=8)
                o0 = ((qid * NREL + r) * N + si * _RS_PER_W
                      + ch * _RS_CHUNK)
                pend[b] = pltpu.async_copy(
                    ob, out_hbm.at[pl.ds(o0, _RS_CHUNK), :], sems[b])
        for b in range(2):
            if pend[b] is not None:
                pend[b].wait()

    return rs_kernel


_rel_scale = _rel_scale_make()


def _mm_body(a0_ref, a1_ref, a2_ref, a3_ref, w_ref, out_ref):
    a = jnp.concatenate(
        [a0_ref[...], a1_ref[...], a2_ref[...], a3_ref[...]], axis=1)
    w = w_ref[...].reshape(DIM, Q)
    r = jnp.dot(a, w, preferred_element_type=jnp.float32)
    out_ref[...] = jnp.maximum(r, 0.0)


def _mm_relu(agg_q, w):
    grid = (4 * _NRB,)
    call = pl.pallas_call(
        _mm_body,
        grid=grid,
        in_specs=[
            pl.BlockSpec((_BLK, Q), lambda j: (0 * _NRB + j % _NRB, 0)),
            pl.BlockSpec((_BLK, Q), lambda j: (1 * _NRB + j % _NRB, 0)),
            pl.BlockSpec((_BLK, Q), lambda j: (2 * _NRB + j % _NRB, 0)),
            pl.BlockSpec((_BLK, Q), lambda j: (3 * _NRB + j % _NRB, 0)),
            pl.BlockSpec((1, DIM, Q), lambda j: (j // _NRB, 0, 0)),
        ],
        out_specs=pl.BlockSpec((_BLK, Q), lambda j: (j, 0)),
        out_shape=jax.ShapeDtypeStruct((4 * N, Q), jnp.float32),
    )
    w4 = jnp.stack([w[:, i * Q:(i + 1) * Q] for i in range(4)])
    return call(agg_q, agg_q, agg_q, agg_q, w4)


_IT_PER_W = B * L // 32
_IT_CHUNK = 400
_U_PER_W = B // 32
_P_PER_W = B * T // 32


def _gather_kernel_make():
    mesh = plsc.VectorSubcoreMesh(core_axis_name="c", subcore_axis_name="s")
    scratch = [
        pltpu.VMEM((_IT_PER_W,), jnp.int32),
        pltpu.VMEM((_IT_CHUNK, Q), jnp.float32),
        pltpu.VMEM((_P_PER_W, AD), jnp.float32),
        pltpu.VMEM((_P_PER_W,), jnp.float32),
        pltpu.SemaphoreType.DMA,
    ]
    out_type = [
        jax.ShapeDtypeStruct((B, AD), jnp.float32),
        jax.ShapeDtypeStruct((B * L, AD), jnp.float32),
        jax.ShapeDtypeStruct((B * T, AD), jnp.float32),
        jax.ShapeDtypeStruct((B * T,), jnp.float32),
    ]

    @functools.partial(pl.kernel, mesh=mesh, out_type=out_type,
                       scratch_types=scratch, compiler_params=_SC_PARAMS)
    def gather_kernel(x1, x2, x3, bu4, seq4, itp, pw, pb,
                      out_u, out_i, out_w, out_b,
                      idxv, rows, prow, pbv, sem):
        cid = lax.axis_index("c")
        sid = lax.axis_index("s")
        w = sid * 2 + cid

        ib = w * _IT_PER_W
        subs = ((0, 128), (128, 128), (256, 128), (384, 16))
        for q in range(4):
            pltpu.sync_copy(seq4.at[q, pl.ds(ib, _IT_PER_W)], idxv)
            for ch in range(4):
                for t, tb in enumerate((x1, x2, x3)):
                    for off, sz in subs:
                        pltpu.async_copy(
                            tb.at[idxv.at[pl.ds(ch * _IT_CHUNK + off, sz)]],
                            rows.at[pl.ds(off, sz), :], sem).wait()
                    pltpu.sync_copy(
                        rows,
                        out_i.at[pl.ds(ib + ch * _IT_CHUNK, _IT_CHUNK),
                                 pl.ds(t * DIM + q * Q, Q)])

        ub = w * _U_PER_W
        for q in range(4):
            pltpu.sync_copy(bu4.at[q, pl.ds(ub, _U_PER_W)],
                            idxv.at[pl.ds(0, _U_PER_W)])
            for t, tb in enumerate((x1, x2, x3)):
                pltpu.async_copy(tb.at[idxv.at[pl.ds(0, _U_PER_W)]],
                                 rows.at[pl.ds(0, _U_PER_W), :], sem).wait()
                pltpu.sync_copy(
                    rows.at[pl.ds(0, _U_PER_W), :],
                    out_u.at[pl.ds(ub, _U_PER_W),
                             pl.ds(t * DIM + q * Q, Q)])

        pbase = w * _P_PER_W
        pltpu.sync_copy(itp.at[pl.ds(pbase, _P_PER_W)],
                        idxv.at[pl.ds(0, _P_PER_W)])
        for off, sz in ((0, 128), (128, 64)):
            pltpu.async_copy(pw.at[idxv.at[pl.ds(off, sz)]],
                             prow.at[pl.ds(off, sz), :], sem).wait()
            pltpu.async_copy(pb.at[idxv.at[pl.ds(off, sz)]],
                             pbv.at[pl.ds(off, sz)], sem).wait()
        pltpu.sync_copy(prow, out_w.at[pl.ds(pbase, _P_PER_W), :])
        pltpu.sync_copy(pbv, out_b.at[pl.ds(pbase, _P_PER_W)])

    return gather_kernel


_gather_rows = _gather_kernel_make()

_NB = 32
_SCALE = 1.0 / math.sqrt(float(AD))


def _attn_body(user_ref, item_ref, pew_ref, peb_ref,
               wq_ref, wk_ref, wv_ref, wcq_ref, wck_ref, wcv_ref,
               res_ref, item_out_ref):
    item = item_ref[...]
    user = user_ref[...]
    q = jnp.dot(item, wq_ref[...], preferred_element_type=jnp.float32)
    k = jnp.dot(item, wk_ref[...], preferred_element_type=jnp.float32)
    v = jnp.dot(item, wv_ref[...], preferred_element_type=jnp.float32)
    q3 = q.reshape(_NB, L, AD)
    k3 = k.reshape(_NB, L, AD)
    v3 = v.reshape(_NB, L, AD)
    s = lax.dot_general(q3, k3, (((2,), (2,)), ((0,), (0,))),
                        preferred_element_type=jnp.float32) * _SCALE
    s = s - jnp.max(s, axis=-1, keepdims=True)
    e = jnp.exp(s)
    attn = e / jnp.sum(e, axis=-1, keepdims=True)
    it3 = item.reshape(_NB, L, AD) + lax.dot_general(
        attn, v3, (((2,), (1,)), ((0,), (0,))),
        preferred_element_type=jnp.float32)
    qc = jnp.dot(it3.reshape(_NB * L, AD), wcq_ref[...],
                 preferred_element_type=jnp.float32).reshape(_NB, L, AD)
    kc = jnp.dot(user, wck_ref[...], preferred_element_type=jnp.float32)
    vc = jnp.dot(user, wcv_ref[...], preferred_element_type=jnp.float32)
    score = jnp.sum(qc * kc[:, None, :], axis=-1) * _SCALE
    score = score - jnp.max(score, axis=-1, keepdims=True)
    es = jnp.exp(score)
    alpha = es / jnp.sum(es, axis=-1, keepdims=True)
    it4 = it3 + alpha[:, :, None] * vc[:, None, :]
    item_out_ref[...] = it4.reshape(_NB * L, AD)
    isum = jnp.sum(it4, axis=1)
    up = user + isum
    pew = pew_ref[...].reshape(_NB, T, AD)
    res_ref[...] = peb_ref[...] + jnp.sum(pew * up[:, None, :], axis=-1)


def _attention(user0, item0, pew, peb, wq, wk, wv, wcq, wck, wcv):
    grid = (B // _NB,)
    wspec = pl.BlockSpec((AD, AD), lambda i: (0, 0))
    return pl.pallas_call(
        _attn_body,
        grid=grid,
        in_specs=[
            pl.BlockSpec((_NB, AD), lambda i: (i, 0)),
            pl.BlockSpec((_NB * L, AD), lambda i: (i, 0)),
            pl.BlockSpec((_NB * T, AD), lambda i: (i, 0)),
            pl.BlockSpec((_NB, T), lambda i: (i, 0)),
            wspec, wspec, wspec, wspec, wspec, wspec,
        ],
        out_specs=[
            pl.BlockSpec((_NB, T), lambda i: (i, 0)),
            pl.BlockSpec((_NB * L, AD), lambda i: (i, 0)),
        ],
        out_shape=[
            jax.ShapeDtypeStruct((B, T), jnp.float32),
            jax.ShapeDtypeStruct((B * L, AD), jnp.float32),
        ],
    )(user0, item0, pew, peb, wq, wk, wv, wcq, wck, wcv)


def _pad_edges(gidx, dst, e_pad, sect_rows):
    e = gidx.shape[0]
    npad = e_pad - e
    padi = jnp.arange(npad, dtype=jnp.int32)
    g_p = jnp.concatenate([gidx, padi % N])
    dst_p = jnp.concatenate([dst, N + (padi % (ACC_ROWS - N))])
    g4 = jnp.stack([g_p + q * sect_rows for q in range(4)])
    return g4, dst_p


def _to_quarters(x):
    m = x.shape[0]
    return x.reshape(m, 4, Q).transpose(1, 0, 2).reshape(4 * m, Q)


def kernel(batch_users, batch_sequences, items_to_predict, edge_index,
           edge_type, node_no, short_edge_index, node_emb, rel_emb,
           W_long0, W_long1, W_short0, Wq, Wk, Wv, Wcq, Wck, Wcv,
           predict_w, predict_b):
    src, dst = edge_index[0], edge_index[1]
    s_src, s_dst = short_edge_index[0], short_edge_index[1]
    gidxL = edge_type * N + src
    g4L, d2L = _pad_edges(gidxL, dst, E_LONG_PAD, 8 * N)
    g4S, d2S = _pad_edges(s_src, s_dst, E_SHORT_PAD, N)

    xq0 = _to_quarters(node_emb)
    rel4 = _to_quarters(rel_emb)

    sc0 = _rel_scale(xq0, rel4)
    agg1 = _segsum_long(g4L, d2L, sc0)
    x1q = _mm_relu(agg1, W_long0)
    sc1 = _rel_scale(x1q, rel4)
    agg2 = _segsum_long(g4L, d2L, sc1)
    x2q = _mm_relu(agg2, W_long1)
    agg3 = _segsum_short(g4S, d2S, x2q)
    x3q = _mm_relu(agg3, W_short0)

    seq = batch_sequences.reshape(-1)
    seq4 = jnp.stack([seq + q * N for q in range(4)])
    bu4 = jnp.stack([batch_users + q * N for q in range(4)])
    user0, item0, pew, peb = _gather_rows(
        x1q, x2q, x3q, bu4, seq4,
        items_to_predict.reshape(-1), predict_w, predict_b[:, 0])

    res, item_out = _attention(user0, item0, pew, peb.reshape(B, T),
                               Wq, Wk, Wv, Wcq, Wck, Wcv)
    return (res, user0, item_out.reshape(B, L, AD))


# speedup vs baseline: 5.0196x; 1.1811x over previous
import functools
import math

import jax
import jax.numpy as jnp
from jax import lax
from jax.experimental import pallas as pl
from jax.experimental.pallas import tpu as pltpu
from jax.experimental.pallas import tpu_sc as plsc

N = 50000
DIM = 64
Q = 16
NREL = 8
AD = 192
B = 1024
L = 50
T = 6
CHUNK = 1024
SUB = 128
ACC_ROWS = N + 48
E_LONG_PAD = 819200
E_SHORT_PAD = 229376
_NTILES = 16

_SC_PARAMS = pltpu.CompilerParams(use_tc_tiling_on_sc=False)


def _make_segsum(n_chunks_per_tile: int):
    mesh = plsc.VectorSubcoreMesh(core_axis_name="c", subcore_axis_name="s")
    scratch = [
        pltpu.VMEM((CHUNK,), jnp.int32),
        pltpu.VMEM((CHUNK,), jnp.int32),
        pltpu.VMEM((CHUNK,), jnp.int32),
        pltpu.VMEM((CHUNK,), jnp.int32),
        pltpu.VMEM((CHUNK, Q), jnp.float32),
        pltpu.VMEM((CHUNK, Q), jnp.float32),
        pltpu.VMEM_SHARED((ACC_ROWS, Q), jnp.float32),
        pltpu.SemaphoreType.DMA,
        pltpu.SemaphoreType.DMA,
    ]
    out_type = jax.ShapeDtypeStruct((4 * N, Q), jnp.float32)
    assert n_chunks_per_tile % 2 == 0

    @functools.partial(pl.kernel, mesh=mesh, out_type=out_type,
                       scratch_types=scratch, compiler_params=_SC_PARAMS)
    def seg_kernel(src4_hbm, dst2_hbm, x_hbm, out_hbm,
                   srcv0, srcv1, dstv0, dstv1, rows0, rows1, acc,
                   sem0, sem1):
        cid = lax.axis_index("c")
        sid = lax.axis_index("s")

        zero16 = jnp.zeros((16,), jnp.float32)

        def zrow0(i, carry):
            rows0[i, pl.ds(0, 16)] = zero16
            return carry

        def chunk_off(j):
            return (sid + j * _NTILES) * CHUNK

        for p in range(2):
            qid = 2 * p + cid
            lax.fori_loop(0, CHUNK, zrow0, 0, unroll=8)
            r0 = sid * 3128
            pltpu.sync_copy(rows0.at[:, :], acc.at[pl.ds(r0, CHUNK), :])
            pltpu.sync_copy(rows0.at[:, :],
                            acc.at[pl.ds(r0 + CHUNK, CHUNK), :])
            pltpu.sync_copy(rows0.at[:, :],
                            acc.at[pl.ds(r0 + 2 * CHUNK, CHUNK), :])
            pltpu.sync_copy(rows0.at[pl.ds(0, 56), :],
                            acc.at[pl.ds(r0 + 3 * CHUNK, 56), :])
            plsc.subcore_barrier()

            nhalf = n_chunks_per_tile // 2
            e0 = chunk_off(0)
            pltpu.sync_copy(src4_hbm.at[qid, pl.ds(e0, CHUNK)], srcv0)
            pltpu.sync_copy(dst2_hbm.at[pl.ds(e0, CHUNK)], dstv0)
            pltpu.async_copy(x_hbm.at[srcv0], rows0, sem0)

            def pair_body(k, carry):
                e1 = chunk_off(2 * k + 1)
                pltpu.sync_copy(src4_hbm.at[qid, pl.ds(e1, CHUNK)], srcv1)
                pltpu.sync_copy(dst2_hbm.at[pl.ds(e1, CHUNK)], dstv1)
                pltpu.async_copy(x_hbm.at[srcv1], rows1, sem1)
                pltpu.make_async_copy(x_hbm.at[srcv0], rows0, sem0).wait()
                pltpu.sync_copy(rows0, acc.at[dstv0], add=True)

                @pl.when(k < nhalf - 1)
                def _prefetch_even():
                    e2 = chunk_off(2 * k + 2)
                    pltpu.sync_copy(src4_hbm.at[qid, pl.ds(e2, CHUNK)],
                                    srcv0)
                    pltpu.sync_copy(dst2_hbm.at[pl.ds(e2, CHUNK)], dstv0)
                    pltpu.async_copy(x_hbm.at[srcv0], rows0, sem0)

                pltpu.make_async_copy(x_hbm.at[srcv1], rows1, sem1).wait()
                pltpu.sync_copy(rows1, acc.at[dstv1], add=True)
                return carry

            lax.fori_loop(0, nhalf, pair_body, 0)
            plsc.subcore_barrier()
            o0 = sid * 3128

            @pl.when(sid < _NTILES - 1)
            def _full_stripe():
                pltpu.sync_copy(acc.at[pl.ds(o0, 3128), :],
                                out_hbm.at[pl.ds(qid * N + o0, 3128), :])

            @pl.when(sid == _NTILES - 1)
            def _last_stripe():
                pltpu.sync_copy(acc.at[pl.ds(o0, 3080), :],
                                out_hbm.at[pl.ds(qid * N + o0, 3080), :])

    return seg_kernel


_segsum_long = _make_segsum(E_LONG_PAD // CHUNK // _NTILES)
_segsum_short = _make_segsum(E_SHORT_PAD // CHUNK // _NTILES)

_BLK = 2000
_NRB = N // _BLK
_RS_CHUNK = 625
_RS_PER_W = N // 8


def _rel_scale_make():
    mesh = plsc.VectorSubcoreMesh(core_axis_name="c", subcore_axis_name="s")
    scratch = [
        pltpu.VMEM((_RS_CHUNK, Q), jnp.float32),
        pltpu.VMEM((_RS_CHUNK, Q), jnp.float32),
        pltpu.VMEM((_RS_CHUNK, Q), jnp.float32),
        pltpu.VMEM((NREL, Q), jnp.float32),
        pltpu.SemaphoreType.DMA,
        pltpu.SemaphoreType.DMA,
    ]
    out_type = jax.ShapeDtypeStruct((32 * N, Q), jnp.float32)

    @functools.partial(pl.kernel, mesh=mesh, out_type=out_type,
                       scratch_types=scratch, compiler_params=_SC_PARAMS)
    def rs_kernel(x_hbm, rel_hbm, out_hbm, xbuf, obuf0, obuf1, relv,
                  sem0, sem1):
        cid = lax.axis_index("c")
        sid = lax.axis_index("s")
        w = sid * 2 + cid
        qid = w // 8
        si = w % 8
        base = qid * N + si * _RS_PER_W
        pltpu.sync_copy(rel_hbm.at[pl.ds(qid * NREL, NREL), :], relv)
        obufs = (obuf0, obuf1)
        sems = (sem0, sem1)
        pend = [None, None]
        for ch in range(_RS_PER_W // _RS_CHUNK):
            pltpu.sync_copy(
                x_hbm.at[pl.ds(base + ch * _RS_CHUNK, _RS_CHUNK), :], xbuf)
            for r in range(NREL):
                b = r % 2
                if pend[b] is not None:
                    pend[b].wait()
                ob, rv = obufs[b], relv[r, pl.ds(0, Q)]

                @plsc.parallel_loop(0, _RS_CHUNK, unroll=8)
                def _mul(i, ob=ob, rv=rv):
                    ob[i, pl.ds(0, Q)] = xbuf[i, pl.ds(0, Q)] * rv
                o0 = ((qid * NREL + r) * N + si * _RS_PER_W
                      + ch * _RS_CHUNK)
                pend[b] = pltpu.async_copy(
                    ob, out_hbm.at[pl.ds(o0, _RS_CHUNK), :], sems[b])
        for b in range(2):
            if pend[b] is not None:
                pend[b].wait()

    return rs_kernel


_rel_scale = _rel_scale_make()


def _mm_body(a0_ref, a1_ref, a2_ref, a3_ref, w_ref, out_ref):
    a = jnp.concatenate(
        [a0_ref[...], a1_ref[...], a2_ref[...], a3_ref[...]], axis=1)
    w = w_ref[...].reshape(DIM, Q)
    r = jnp.dot(a, w, preferred_element_type=jnp.float32)
    out_ref[...] = jnp.maximum(r, 0.0)


def _mm_relu(agg_q, w):
    grid = (4 * _NRB,)
    call = pl.pallas_call(
        _mm_body,
        grid=grid,
        in_specs=[
            pl.BlockSpec((_BLK, Q), lambda j: (0 * _NRB + j % _NRB, 0)),
            pl.BlockSpec((_BLK, Q), lambda j: (1 * _NRB + j % _NRB, 0)),
            pl.BlockSpec((_BLK, Q), lambda j: (2 * _NRB + j % _NRB, 0)),
            pl.BlockSpec((_BLK, Q), lambda j: (3 * _NRB + j % _NRB, 0)),
            pl.BlockSpec((1, DIM, Q), lambda j: (j // _NRB, 0, 0)),
        ],
        out_specs=pl.BlockSpec((_BLK, Q), lambda j: (j, 0)),
        out_shape=jax.ShapeDtypeStruct((4 * N, Q), jnp.float32),
    )
    w4 = jnp.stack([w[:, i * Q:(i + 1) * Q] for i in range(4)])
    return call(agg_q, agg_q, agg_q, agg_q, w4)


_IT_PER_W = B * L // 32
_IT_CHUNK = 400
_U_PER_W = B // 32
_P_PER_W = B * T // 32


def _gather_kernel_make():
    mesh = plsc.VectorSubcoreMesh(core_axis_name="c", subcore_axis_name="s")
    scratch = [
        pltpu.VMEM((_IT_PER_W,), jnp.int32),
        pltpu.VMEM((_IT_CHUNK, Q), jnp.float32),
        pltpu.VMEM((_P_PER_W, AD), jnp.float32),
        pltpu.VMEM((_P_PER_W,), jnp.float32),
        pltpu.SemaphoreType.DMA,
    ]
    out_type = [
        jax.ShapeDtypeStruct((B, AD), jnp.float32),
        jax.ShapeDtypeStruct((B * L, AD), jnp.float32),
        jax.ShapeDtypeStruct((B * T, AD), jnp.float32),
        jax.ShapeDtypeStruct((B * T,), jnp.float32),
    ]

    @functools.partial(pl.kernel, mesh=mesh, out_type=out_type,
                       scratch_types=scratch, compiler_params=_SC_PARAMS)
    def gather_kernel(x1, x2, x3, bu4, seq4, itp, pw, pb,
                      out_u, out_i, out_w, out_b,
                      idxv, rows, prow, pbv, sem):
        cid = lax.axis_index("c")
        sid = lax.axis_index("s")
        w = sid * 2 + cid

        ib = w * _IT_PER_W
        subs = ((0, 128), (128, 128), (256, 128), (384, 16))
        for q in range(4):
            pltpu.sync_copy(seq4.at[q, pl.ds(ib, _IT_PER_W)], idxv)
            for ch in range(4):
                for t, tb in enumerate((x1, x2, x3)):
                    for off, sz in subs:
                        pltpu.async_copy(
                            tb.at[idxv.at[pl.ds(ch * _IT_CHUNK + off, sz)]],
                            rows.at[pl.ds(off, sz), :], sem).wait()
                    pltpu.sync_copy(
                        rows,
                        out_i.at[pl.ds(ib + ch * _IT_CHUNK, _IT_CHUNK),
                                 pl.ds(t * DIM + q * Q, Q)])

        ub = w * _U_PER_W
        for q in range(4):
            pltpu.sync_copy(bu4.at[q, pl.ds(ub, _U_PER_W)],
                            idxv.at[pl.ds(0, _U_PER_W)])
            for t, tb in enumerate((x1, x2, x3)):
                pltpu.async_copy(tb.at[idxv.at[pl.ds(0, _U_PER_W)]],
                                 rows.at[pl.ds(0, _U_PER_W), :], sem).wait()
                pltpu.sync_copy(
                    rows.at[pl.ds(0, _U_PER_W), :],
                    out_u.at[pl.ds(ub, _U_PER_W),
                             pl.ds(t * DIM + q * Q, Q)])

        pbase = w * _P_PER_W
        pltpu.sync_copy(itp.at[pl.ds(pbase, _P_PER_W)],
                        idxv.at[pl.ds(0, _P_PER_W)])
        for off, sz in ((0, 128), (128, 64)):
            pltpu.async_copy(pw.at[idxv.at[pl.ds(off, sz)]],
                             prow.at[pl.ds(off, sz), :], sem).wait()
            pltpu.async_copy(pb.at[idxv.at[pl.ds(off, sz)]],
                             pbv.at[pl.ds(off, sz)], sem).wait()
        pltpu.sync_copy(prow, out_w.at[pl.ds(pbase, _P_PER_W), :])
        pltpu.sync_copy(pbv, out_b.at[pl.ds(pbase, _P_PER_W)])

    return gather_kernel


_gather_rows = _gather_kernel_make()

_NB = 32
_SCALE = 1.0 / math.sqrt(float(AD))


def _attn_body(user_ref, item_ref, pew_ref, peb_ref,
               wq_ref, wk_ref, wv_ref, wcq_ref, wck_ref, wcv_ref,
               res_ref, item_out_ref):
    item = item_ref[...]
    user = user_ref[...]
    q = jnp.dot(item, wq_ref[...], preferred_element_type=jnp.float32)
    k = jnp.dot(item, wk_ref[...], preferred_element_type=jnp.float32)
    v = jnp.dot(item, wv_ref[...], preferred_element_type=jnp.float32)
    q3 = q.reshape(_NB, L, AD)
    k3 = k.reshape(_NB, L, AD)
    v3 = v.reshape(_NB, L, AD)
    s = lax.dot_general(q3, k3, (((2,), (2,)), ((0,), (0,))),
                        preferred_element_type=jnp.float32) * _SCALE
    s = s - jnp.max(s, axis=-1, keepdims=True)
    e = jnp.exp(s)
    attn = e / jnp.sum(e, axis=-1, keepdims=True)
    it3 = item.reshape(_NB, L, AD) + lax.dot_general(
        attn, v3, (((2,), (1,)), ((0,), (0,))),
        preferred_element_type=jnp.float32)
    qc = jnp.dot(it3.reshape(_NB * L, AD), wcq_ref[...],
                 preferred_element_type=jnp.float32).reshape(_NB, L, AD)
    kc = jnp.dot(user, wck_ref[...], preferred_element_type=jnp.float32)
    vc = jnp.dot(user, wcv_ref[...], preferred_element_type=jnp.float32)
    score = jnp.sum(qc * kc[:, None, :], axis=-1) * _SCALE
    score = score - jnp.max(score, axis=-1, keepdims=True)
    es = jnp.exp(score)
    alpha = es / jnp.sum(es, axis=-1, keepdims=True)
    it4 = it3 + alpha[:, :, None] * vc[:, None, :]
    item_out_ref[...] = it4.reshape(_NB * L, AD)
    isum = jnp.sum(it4, axis=1)
    up = user + isum
    pew = pew_ref[...].reshape(_NB, T, AD)
    res_ref[...] = peb_ref[...] + jnp.sum(pew * up[:, None, :], axis=-1)


def _attention(user0, item0, pew, peb, wq, wk, wv, wcq, wck, wcv):
    grid = (B // _NB,)
    wspec = pl.BlockSpec((AD, AD), lambda i: (0, 0))
    return pl.pallas_call(
        _attn_body,
        grid=grid,
        in_specs=[
            pl.BlockSpec((_NB, AD), lambda i: (i, 0)),
            pl.BlockSpec((_NB * L, AD), lambda i: (i, 0)),
            pl.BlockSpec((_NB * T, AD), lambda i: (i, 0)),
            pl.BlockSpec((_NB, T), lambda i: (i, 0)),
            wspec, wspec, wspec, wspec, wspec, wspec,
        ],
        out_specs=[
            pl.BlockSpec((_NB, T), lambda i: (i, 0)),
            pl.BlockSpec((_NB * L, AD), lambda i: (i, 0)),
        ],
        out_shape=[
            jax.ShapeDtypeStruct((B, T), jnp.float32),
            jax.ShapeDtypeStruct((B * L, AD), jnp.float32),
        ],
    )(user0, item0, pew, peb, wq, wk, wv, wcq, wck, wcv)


def _pad_edges(gidx, dst, e_pad, sect_rows):
    e = gidx.shape[0]
    npad = e_pad - e
    padi = jnp.arange(npad, dtype=jnp.int32)
    g_p = jnp.concatenate([gidx, padi % N])
    dst_p = jnp.concatenate([dst, N + (padi % (ACC_ROWS - N))])
    g4 = jnp.stack([g_p + q * sect_rows for q in range(4)])
    return g4, dst_p


def _to_quarters(x):
    m = x.shape[0]
    return x.reshape(m, 4, Q).transpose(1, 0, 2).reshape(4 * m, Q)


def kernel(batch_users, batch_sequences, items_to_predict, edge_index,
           edge_type, node_no, short_edge_index, node_emb, rel_emb,
           W_long0, W_long1, W_short0, Wq, Wk, Wv, Wcq, Wck, Wcv,
           predict_w, predict_b):
    src, dst = edge_index[0], edge_index[1]
    s_src, s_dst = short_edge_index[0], short_edge_index[1]
    gidxL = edge_type * N + src
    g4L, d2L = _pad_edges(gidxL, dst, E_LONG_PAD, 8 * N)
    g4S, d2S = _pad_edges(s_src, s_dst, E_SHORT_PAD, N)

    xq0 = _to_quarters(node_emb)
    rel4 = _to_quarters(rel_emb)

    sc0 = _rel_scale(xq0, rel4)
    agg1 = _segsum_long(g4L, d2L, sc0)
    x1q = _mm_relu(agg1, W_long0)
    sc1 = _rel_scale(x1q, rel4)
    agg2 = _segsum_long(g4L, d2L, sc1)
    x2q = _mm_relu(agg2, W_long1)
    agg3 = _segsum_short(g4S, d2S, x2q)
    x3q = _mm_relu(agg3, W_short0)

    seq = batch_sequences.reshape(-1)
    seq4 = jnp.stack([seq + q * N for q in range(4)])
    bu4 = jnp.stack([batch_users + q * N for q in range(4)])
    user0, item0, pew, peb = _gather_rows(
        x1q, x2q, x3q, bu4, seq4,
        items_to_predict.reshape(-1), predict_w, predict_b[:, 0])

    res, item_out = _attention(user0, item0, pew, peb.reshape(B, T),
                               Wq, Wk, Wv, Wcq, Wck, Wcv)
    return (res, user0, item_out.reshape(B, L, AD))


# speedup vs baseline: 5.0545x; 1.0069x over previous
import functools
import math

import jax
import jax.numpy as jnp
from jax import lax
from jax.experimental import pallas as pl
from jax.experimental.pallas import tpu as pltpu
from jax.experimental.pallas import tpu_sc as plsc

N = 50000
DIM = 64
Q = 16
NREL = 8
AD = 192
B = 1024
L = 50
T = 6
CHUNK = 1024
SUB = 128
ACC_ROWS = N + 48
E_LONG_PAD = 819200
E_SHORT_PAD = 229376
_NTILES = 16

_SC_PARAMS = pltpu.CompilerParams(use_tc_tiling_on_sc=False)


def _make_segsum(n_chunks_per_tile: int):
    mesh = plsc.VectorSubcoreMesh(core_axis_name="c", subcore_axis_name="s")
    scratch = [
        pltpu.VMEM((CHUNK,), jnp.int32),
        pltpu.VMEM((CHUNK,), jnp.int32),
        pltpu.VMEM((CHUNK,), jnp.int32),
        pltpu.VMEM((CHUNK,), jnp.int32),
        pltpu.VMEM((CHUNK, Q), jnp.float32),
        pltpu.VMEM((CHUNK, Q), jnp.float32),
        pltpu.VMEM_SHARED((ACC_ROWS, Q), jnp.float32),
        pltpu.SemaphoreType.DMA,
        pltpu.SemaphoreType.DMA,
    ]
    out_type = jax.ShapeDtypeStruct((4 * N, Q), jnp.float32)
    assert n_chunks_per_tile % 2 == 0

    @functools.partial(pl.kernel, mesh=mesh, out_type=out_type,
                       scratch_types=scratch, compiler_params=_SC_PARAMS)
    def seg_kernel(src4_hbm, dst2_hbm, x_hbm, out_hbm,
                   srcv0, srcv1, dstv0, dstv1, rows0, rows1, acc,
                   sem0, sem1):
        cid = lax.axis_index("c")
        sid = lax.axis_index("s")

        zero16 = jnp.zeros((16,), jnp.float32)

        def chunk_off(j):
            return (sid + j * _NTILES) * CHUNK

        for p in range(2):
            qid = 2 * p + cid

            @plsc.parallel_loop(0, CHUNK, unroll=8)
            def _zrow(i):
                rows0[i, pl.ds(0, 16)] = zero16

            r0 = sid * 3128
            pltpu.sync_copy(rows0.at[:, :], acc.at[pl.ds(r0, CHUNK), :])
            pltpu.sync_copy(rows0.at[:, :],
                            acc.at[pl.ds(r0 + CHUNK, CHUNK), :])
            pltpu.sync_copy(rows0.at[:, :],
                            acc.at[pl.ds(r0 + 2 * CHUNK, CHUNK), :])
            pltpu.sync_copy(rows0.at[pl.ds(0, 56), :],
                            acc.at[pl.ds(r0 + 3 * CHUNK, 56), :])
            plsc.subcore_barrier()

            nhalf = n_chunks_per_tile // 2
            e0 = chunk_off(0)
            pltpu.sync_copy(src4_hbm.at[qid, pl.ds(e0, CHUNK)], srcv0)
            pltpu.sync_copy(dst2_hbm.at[pl.ds(e0, CHUNK)], dstv0)
            pltpu.async_copy(x_hbm.at[srcv0], rows0, sem0)

            def pair_body(k, carry):
                e1 = chunk_off(2 * k + 1)
                pltpu.sync_copy(src4_hbm.at[qid, pl.ds(e1, CHUNK)], srcv1)
                pltpu.sync_copy(dst2_hbm.at[pl.ds(e1, CHUNK)], dstv1)
                pltpu.async_copy(x_hbm.at[srcv1], rows1, sem1)
                pltpu.make_async_copy(x_hbm.at[srcv0], rows0, sem0).wait()
                pltpu.sync_copy(rows0, acc.at[dstv0], add=True)

                @pl.when(k < nhalf - 1)
                def _prefetch_even():
                    e2 = chunk_off(2 * k + 2)
                    pltpu.sync_copy(src4_hbm.at[qid, pl.ds(e2, CHUNK)],
                                    srcv0)
                    pltpu.sync_copy(dst2_hbm.at[pl.ds(e2, CHUNK)], dstv0)
                    pltpu.async_copy(x_hbm.at[srcv0], rows0, sem0)

                pltpu.make_async_copy(x_hbm.at[srcv1], rows1, sem1).wait()
                pltpu.sync_copy(rows1, acc.at[dstv1], add=True)
                return carry

            lax.fori_loop(0, nhalf, pair_body, 0)
            plsc.subcore_barrier()
            o0 = sid * 3128

            @pl.when(sid < _NTILES - 1)
            def _full_stripe():
                pltpu.sync_copy(acc.at[pl.ds(o0, 3128), :],
                                out_hbm.at[pl.ds(qid * N + o0, 3128), :])

            @pl.when(sid == _NTILES - 1)
            def _last_stripe():
                pltpu.sync_copy(acc.at[pl.ds(o0, 3080), :],
                                out_hbm.at[pl.ds(qid * N + o0, 3080), :])

    return seg_kernel


_segsum_long = _make_segsum(E_LONG_PAD // CHUNK // _NTILES)
_segsum_short = _make_segsum(E_SHORT_PAD // CHUNK // _NTILES)

_BLK = 2000
_NRB = N // _BLK
_RS_CHUNK = 625
_RS_PER_W = N // 8


def _rel_scale_make():
    mesh = plsc.VectorSubcoreMesh(core_axis_name="c", subcore_axis_name="s")
    scratch = [
        pltpu.VMEM((_RS_CHUNK, Q), jnp.float32),
        pltpu.VMEM((_RS_CHUNK, Q), jnp.float32),
        pltpu.VMEM((_RS_CHUNK, Q), jnp.float32),
        pltpu.VMEM((_RS_CHUNK, Q), jnp.float32),
        pltpu.VMEM((NREL, Q), jnp.float32),
        pltpu.SemaphoreType.DMA,
        pltpu.SemaphoreType.DMA,
        pltpu.SemaphoreType.DMA,
    ]
    out_type = jax.ShapeDtypeStruct((32 * N, Q), jnp.float32)
    n_ch = _RS_PER_W // _RS_CHUNK

    @functools.partial(pl.kernel, mesh=mesh, out_type=out_type,
                       scratch_types=scratch, compiler_params=_SC_PARAMS)
    def rs_kernel(x_hbm, rel_hbm, out_hbm, xbuf0, xbuf1, obuf0, obuf1,
                  relv, sem0, sem1, semx):
        cid = lax.axis_index("c")
        sid = lax.axis_index("s")
        w = sid * 2 + cid
        qid = w // 8
        si = w % 8
        base = qid * N + si * _RS_PER_W
        pltpu.sync_copy(rel_hbm.at[pl.ds(qid * NREL, NREL), :], relv)
        obufs = (obuf0, obuf1)
        xbufs = (xbuf0, xbuf1)
        sems = (sem0, sem1)
        pend = [None, None]
        xpend = pltpu.async_copy(
            x_hbm.at[pl.ds(base, _RS_CHUNK), :], xbuf0, semx)
        for ch in range(n_ch):
            xpend.wait()
            xbuf = xbufs[ch % 2]
            if ch + 1 < n_ch:
                xpend = pltpu.async_copy(
                    x_hbm.at[pl.ds(base + (ch + 1) * _RS_CHUNK, _RS_CHUNK),
                             :], xbufs[(ch + 1) % 2], semx)
            for r in range(NREL):
                b = r % 2
                if pend[b] is not None:
                    pend[b].wait()
                ob, rv = obufs[b], relv[r, pl.ds(0, Q)]

                @plsc.parallel_loop(0, _RS_CHUNK, unroll=8)
                def _mul(i, ob=ob, rv=rv, xb=xbuf):
                    ob[i, pl.ds(0, Q)] = xb[i, pl.ds(0, Q)] * rv
                o0 = ((qid * NREL + r) * N + si * _RS_PER_W
                      + ch * _RS_CHUNK)
                pend[b] = pltpu.async_copy(
                    ob, out_hbm.at[pl.ds(o0, _RS_CHUNK), :], sems[b])
        for b in range(2):
            if pend[b] is not None:
                pend[b].wait()

    return rs_kernel


_rel_scale = _rel_scale_make()


def _mm_body(a0_ref, a1_ref, a2_ref, a3_ref, w_ref, out_ref):
    a = jnp.concatenate(
        [a0_ref[...], a1_ref[...], a2_ref[...], a3_ref[...]], axis=1)
    w = w_ref[...].reshape(DIM, Q)
    r = jnp.dot(a, w, preferred_element_type=jnp.float32)
    out_ref[...] = jnp.maximum(r, 0.0)


def _mm_relu(agg_q, w):
    grid = (4 * _NRB,)
    call = pl.pallas_call(
        _mm_body,
        grid=grid,
        in_specs=[
            pl.BlockSpec((_BLK, Q), lambda j: (0 * _NRB + j % _NRB, 0)),
            pl.BlockSpec((_BLK, Q), lambda j: (1 * _NRB + j % _NRB, 0)),
            pl.BlockSpec((_BLK, Q), lambda j: (2 * _NRB + j % _NRB, 0)),
            pl.BlockSpec((_BLK, Q), lambda j: (3 * _NRB + j % _NRB, 0)),
            pl.BlockSpec((1, DIM, Q), lambda j: (j // _NRB, 0, 0)),
        ],
        out_specs=pl.BlockSpec((_BLK, Q), lambda j: (j, 0)),
        out_shape=jax.ShapeDtypeStruct((4 * N, Q), jnp.float32),
    )
    w4 = jnp.stack([w[:, i * Q:(i + 1) * Q] for i in range(4)])
    return call(agg_q, agg_q, agg_q, agg_q, w4)


_IT_PER_W = B * L // 32
_IT_CHUNK = 400
_U_PER_W = B // 32
_P_PER_W = B * T // 32


def _gather_kernel_make():
    mesh = plsc.VectorSubcoreMesh(core_axis_name="c", subcore_axis_name="s")
    scratch = [
        pltpu.VMEM((_IT_PER_W,), jnp.int32),
        pltpu.VMEM((_IT_CHUNK, Q), jnp.float32),
        pltpu.VMEM((_P_PER_W, AD), jnp.float32),
        pltpu.VMEM((_P_PER_W,), jnp.float32),
        pltpu.SemaphoreType.DMA,
    ]
    out_type = [
        jax.ShapeDtypeStruct((B, AD), jnp.float32),
        jax.ShapeDtypeStruct((B * L, AD), jnp.float32),
        jax.ShapeDtypeStruct((B * T, AD), jnp.float32),
        jax.ShapeDtypeStruct((B * T,), jnp.float32),
    ]

    @functools.partial(pl.kernel, mesh=mesh, out_type=out_type,
                       scratch_types=scratch, compiler_params=_SC_PARAMS)
    def gather_kernel(x1, x2, x3, bu4, seq4, itp, pw, pb,
                      out_u, out_i, out_w, out_b,
                      idxv, rows, prow, pbv, sem):
        cid = lax.axis_index("c")
        sid = lax.axis_index("s")
        w = sid * 2 + cid

        ib = w * _IT_PER_W
        subs = ((0, 128), (128, 128), (256, 128), (384, 16))
        for q in range(4):
            pltpu.sync_copy(seq4.at[q, pl.ds(ib, _IT_PER_W)], idxv)
            for ch in range(4):
                for t, tb in enumerate((x1, x2, x3)):
                    for off, sz in subs:
                        pltpu.async_copy(
                            tb.at[idxv.at[pl.ds(ch * _IT_CHUNK + off, sz)]],
                            rows.at[pl.ds(off, sz), :], sem).wait()
                    pltpu.sync_copy(
                        rows,
                        out_i.at[pl.ds(ib + ch * _IT_CHUNK, _IT_CHUNK),
                                 pl.ds(t * DIM + q * Q, Q)])

        ub = w * _U_PER_W
        for q in range(4):
            pltpu.sync_copy(bu4.at[q, pl.ds(ub, _U_PER_W)],
                            idxv.at[pl.ds(0, _U_PER_W)])
            for t, tb in enumerate((x1, x2, x3)):
                pltpu.async_copy(tb.at[idxv.at[pl.ds(0, _U_PER_W)]],
                                 rows.at[pl.ds(0, _U_PER_W), :], sem).wait()
                pltpu.sync_copy(
                    rows.at[pl.ds(0, _U_PER_W), :],
                    out_u.at[pl.ds(ub, _U_PER_W),
                             pl.ds(t * DIM + q * Q, Q)])

        pbase = w * _P_PER_W
        pltpu.sync_copy(itp.at[pl.ds(pbase, _P_PER_W)],
                        idxv.at[pl.ds(0, _P_PER_W)])
        for off, sz in ((0, 128), (128, 64)):
            pltpu.async_copy(pw.at[idxv.at[pl.ds(off, sz)]],
                             prow.at[pl.ds(off, sz), :], sem).wait()
            pltpu.async_copy(pb.at[idxv.at[pl.ds(off, sz)]],
                             pbv.at[pl.ds(off, sz)], sem).wait()
        pltpu.sync_copy(prow, out_w.at[pl.ds(pbase, _P_PER_W), :])
        pltpu.sync_copy(pbv, out_b.at[pl.ds(pbase, _P_PER_W)])

    return gather_kernel


_gather_rows = _gather_kernel_make()

_NB = 32
_SCALE = 1.0 / math.sqrt(float(AD))


def _attn_body(user_ref, item_ref, pew_ref, peb_ref,
               wq_ref, wk_ref, wv_ref, wcq_ref, wck_ref, wcv_ref,
               res_ref, item_out_ref):
    item = item_ref[...]
    user = user_ref[...]
    q = jnp.dot(item, wq_ref[...], preferred_element_type=jnp.float32)
    k = jnp.dot(item, wk_ref[...], preferred_element_type=jnp.float32)
    v = jnp.dot(item, wv_ref[...], preferred_element_type=jnp.float32)
    q3 = q.reshape(_NB, L, AD)
    k3 = k.reshape(_NB, L, AD)
    v3 = v.reshape(_NB, L, AD)
    s = lax.dot_general(q3, k3, (((2,), (2,)), ((0,), (0,))),
                        preferred_element_type=jnp.float32) * _SCALE
    s = s - jnp.max(s, axis=-1, keepdims=True)
    e = jnp.exp(s)
    attn = e / jnp.sum(e, axis=-1, keepdims=True)
    it3 = item.reshape(_NB, L, AD) + lax.dot_general(
        attn, v3, (((2,), (1,)), ((0,), (0,))),
        preferred_element_type=jnp.float32)
    qc = jnp.dot(it3.reshape(_NB * L, AD), wcq_ref[...],
                 preferred_element_type=jnp.float32).reshape(_NB, L, AD)
    kc = jnp.dot(user, wck_ref[...], preferred_element_type=jnp.float32)
    vc = jnp.dot(user, wcv_ref[...], preferred_element_type=jnp.float32)
    score = jnp.sum(qc * kc[:, None, :], axis=-1) * _SCALE
    score = score - jnp.max(score, axis=-1, keepdims=True)
    es = jnp.exp(score)
    alpha = es / jnp.sum(es, axis=-1, keepdims=True)
    it4 = it3 + alpha[:, :, None] * vc[:, None, :]
    item_out_ref[...] = it4.reshape(_NB * L, AD)
    isum = jnp.sum(it4, axis=1)
    up = user + isum
    pew = pew_ref[...].reshape(_NB, T, AD)
    res_ref[...] = peb_ref[...] + jnp.sum(pew * up[:, None, :], axis=-1)


def _attention(user0, item0, pew, peb, wq, wk, wv, wcq, wck, wcv):
    grid = (B // _NB,)
    wspec = pl.BlockSpec((AD, AD), lambda i: (0, 0))
    return pl.pallas_call(
        _attn_body,
        grid=grid,
        in_specs=[
            pl.BlockSpec((_NB, AD), lambda i: (i, 0)),
            pl.BlockSpec((_NB * L, AD), lambda i: (i, 0)),
            pl.BlockSpec((_NB * T, AD), lambda i: (i, 0)),
            pl.BlockSpec((_NB, T), lambda i: (i, 0)),
            wspec, wspec, wspec, wspec, wspec, wspec,
        ],
        out_specs=[
            pl.BlockSpec((_NB, T), lambda i: (i, 0)),
            pl.BlockSpec((_NB * L, AD), lambda i: (i, 0)),
        ],
        out_shape=[
            jax.ShapeDtypeStruct((B, T), jnp.float32),
            jax.ShapeDtypeStruct((B * L, AD), jnp.float32),
        ],
    )(user0, item0, pew, peb, wq, wk, wv, wcq, wck, wcv)


def _pad_edges(gidx, dst, e_pad, sect_rows):
    e = gidx.shape[0]
    npad = e_pad - e
    padi = jnp.arange(npad, dtype=jnp.int32)
    g_p = jnp.concatenate([gidx, padi % N])
    dst_p = jnp.concatenate([dst, N + (padi % (ACC_ROWS - N))])
    g4 = jnp.stack([g_p + q * sect_rows for q in range(4)])
    return g4, dst_p


def _to_quarters(x):
    m = x.shape[0]
    return x.reshape(m, 4, Q).transpose(1, 0, 2).reshape(4 * m, Q)


def kernel(batch_users, batch_sequences, items_to_predict, edge_index,
           edge_type, node_no, short_edge_index, node_emb, rel_emb,
           W_long0, W_long1, W_short0, Wq, Wk, Wv, Wcq, Wck, Wcv,
           predict_w, predict_b):
    src, dst = edge_index[0], edge_index[1]
    s_src, s_dst = short_edge_index[0], short_edge_index[1]
    gidxL = edge_type * N + src
    g4L, d2L = _pad_edges(gidxL, dst, E_LONG_PAD, 8 * N)
    g4S, d2S = _pad_edges(s_src, s_dst, E_SHORT_PAD, N)

    xq0 = _to_quarters(node_emb)
    rel4 = _to_quarters(rel_emb)

    sc0 = _rel_scale(xq0, rel4)
    agg1 = _segsum_long(g4L, d2L, sc0)
    x1q = _mm_relu(agg1, W_long0)
    sc1 = _rel_scale(x1q, rel4)
    agg2 = _segsum_long(g4L, d2L, sc1)
    x2q = _mm_relu(agg2, W_long1)
    agg3 = _segsum_short(g4S, d2S, x2q)
    x3q = _mm_relu(agg3, W_short0)

    seq = batch_sequences.reshape(-1)
    seq4 = jnp.stack([seq + q * N for q in range(4)])
    bu4 = jnp.stack([batch_users + q * N for q in range(4)])
    user0, item0, pew, peb = _gather_rows(
        x1q, x2q, x3q, bu4, seq4,
        items_to_predict.reshape(-1), predict_w, predict_b[:, 0])

    res, item_out = _attention(user0, item0, pew, peb.reshape(B, T),
                               Wq, Wk, Wv, Wcq, Wck, Wcv)
    return (res, user0, item_out.reshape(B, L, AD))


# speedup vs baseline: 5.0747x; 1.0040x over previous
import functools
import math

import jax
import jax.numpy as jnp
from jax import lax
from jax.experimental import pallas as pl
from jax.experimental.pallas import tpu as pltpu
from jax.experimental.pallas import tpu_sc as plsc

N = 50000
DIM = 64
Q = 16
NREL = 8
AD = 192
B = 1024
L = 50
T = 6
CHUNK = 1024
SUB = 128
ACC_ROWS = N + 48
E_LONG_PAD = 819200
E_SHORT_PAD = 229376
_NTILES = 16

_SC_PARAMS = pltpu.CompilerParams(use_tc_tiling_on_sc=False)


def _make_segsum(n_chunks_per_tile: int):
    mesh = plsc.VectorSubcoreMesh(core_axis_name="c", subcore_axis_name="s")
    scratch = [
        pltpu.VMEM((CHUNK,), jnp.int32),
        pltpu.VMEM((CHUNK,), jnp.int32),
        pltpu.VMEM((CHUNK,), jnp.int32),
        pltpu.VMEM((CHUNK,), jnp.int32),
        pltpu.VMEM((CHUNK, Q), jnp.float32),
        pltpu.VMEM((CHUNK, Q), jnp.float32),
        pltpu.VMEM_SHARED((ACC_ROWS, Q), jnp.float32),
        pltpu.SemaphoreType.DMA,
        pltpu.SemaphoreType.DMA,
    ]
    out_type = jax.ShapeDtypeStruct((4 * N, Q), jnp.float32)
    assert n_chunks_per_tile % 2 == 0

    @functools.partial(pl.kernel, mesh=mesh, out_type=out_type,
                       scratch_types=scratch, compiler_params=_SC_PARAMS)
    def seg_kernel(src4_hbm, dst2_hbm, x_hbm, out_hbm,
                   srcv0, srcv1, dstv0, dstv1, rows0, rows1, acc,
                   sem0, sem1):
        cid = lax.axis_index("c")
        sid = lax.axis_index("s")

        zero16 = jnp.zeros((16,), jnp.float32)

        def chunk_off(j):
            return (sid + j * _NTILES) * CHUNK

        for p in range(2):
            qid = 2 * p + cid

            @plsc.parallel_loop(0, CHUNK, unroll=8)
            def _zrow(i):
                rows0[i, pl.ds(0, 16)] = zero16

            r0 = sid * 3128
            pltpu.sync_copy(rows0.at[:, :], acc.at[pl.ds(r0, CHUNK), :])
            pltpu.sync_copy(rows0.at[:, :],
                            acc.at[pl.ds(r0 + CHUNK, CHUNK), :])
            pltpu.sync_copy(rows0.at[:, :],
                            acc.at[pl.ds(r0 + 2 * CHUNK, CHUNK), :])
            pltpu.sync_copy(rows0.at[pl.ds(0, 56), :],
                            acc.at[pl.ds(r0 + 3 * CHUNK, 56), :])
            plsc.subcore_barrier()

            nhalf = n_chunks_per_tile // 2
            e0 = chunk_off(0)
            pltpu.sync_copy(src4_hbm.at[qid, pl.ds(e0, CHUNK)], srcv0)
            pltpu.sync_copy(dst2_hbm.at[pl.ds(e0, CHUNK)], dstv0)
            pltpu.async_copy(x_hbm.at[srcv0], rows0, sem0)

            def pair_body(k, carry):
                e1 = chunk_off(2 * k + 1)
                pltpu.sync_copy(src4_hbm.at[qid, pl.ds(e1, CHUNK)], srcv1)
                pltpu.sync_copy(dst2_hbm.at[pl.ds(e1, CHUNK)], dstv1)
                pltpu.async_copy(x_hbm.at[srcv1], rows1, sem1)
                pltpu.make_async_copy(x_hbm.at[srcv0], rows0, sem0).wait()
                pltpu.sync_copy(rows0, acc.at[dstv0], add=True)

                @pl.when(k < nhalf - 1)
                def _prefetch_even():
                    e2 = chunk_off(2 * k + 2)
                    pltpu.sync_copy(src4_hbm.at[qid, pl.ds(e2, CHUNK)],
                                    srcv0)
                    pltpu.sync_copy(dst2_hbm.at[pl.ds(e2, CHUNK)], dstv0)
                    pltpu.async_copy(x_hbm.at[srcv0], rows0, sem0)

                pltpu.make_async_copy(x_hbm.at[srcv1], rows1, sem1).wait()
                pltpu.sync_copy(rows1, acc.at[dstv1], add=True)
                return carry

            lax.fori_loop(0, nhalf, pair_body, 0)
            plsc.subcore_barrier()
            o0 = sid * 3128

            @pl.when(sid < _NTILES - 1)
            def _full_stripe():
                pltpu.sync_copy(acc.at[pl.ds(o0, 3128), :],
                                out_hbm.at[pl.ds(qid * N + o0, 3128), :])

            @pl.when(sid == _NTILES - 1)
            def _last_stripe():
                pltpu.sync_copy(acc.at[pl.ds(o0, 3080), :],
                                out_hbm.at[pl.ds(qid * N + o0, 3080), :])

    return seg_kernel


_segsum_long = _make_segsum(E_LONG_PAD // CHUNK // _NTILES)
_segsum_short = _make_segsum(E_SHORT_PAD // CHUNK // _NTILES)

_BLK = 2000
_NRB = N // _BLK
_RS_CHUNK = 625
_RS_PER_W = N // 8


def _rel_scale_make():
    mesh = plsc.VectorSubcoreMesh(core_axis_name="c", subcore_axis_name="s")
    scratch = [
        pltpu.VMEM((_RS_CHUNK, Q), jnp.float32),
        pltpu.VMEM((_RS_CHUNK, Q), jnp.float32),
        pltpu.VMEM((_RS_CHUNK, Q), jnp.float32),
        pltpu.VMEM((_RS_CHUNK, Q), jnp.float32),
        pltpu.VMEM((NREL, Q), jnp.float32),
        pltpu.SemaphoreType.DMA,
        pltpu.SemaphoreType.DMA,
        pltpu.SemaphoreType.DMA,
    ]
    out_type = jax.ShapeDtypeStruct((32 * N, Q), jnp.float32)
    n_ch = _RS_PER_W // _RS_CHUNK

    @functools.partial(pl.kernel, mesh=mesh, out_type=out_type,
                       scratch_types=scratch, compiler_params=_SC_PARAMS)
    def rs_kernel(x_hbm, rel_hbm, out_hbm, xbuf0, xbuf1, obuf0, obuf1,
                  relv, sem0, sem1, semx):
        cid = lax.axis_index("c")
        sid = lax.axis_index("s")
        w = sid * 2 + cid
        qid = w // 8
        si = w % 8
        base = qid * N + si * _RS_PER_W
        pltpu.sync_copy(rel_hbm.at[pl.ds(qid * NREL, NREL), :], relv)
        obufs = (obuf0, obuf1)
        xbufs = (xbuf0, xbuf1)
        sems = (sem0, sem1)
        pend = [None, None]
        xpend = pltpu.async_copy(
            x_hbm.at[pl.ds(base, _RS_CHUNK), :], xbuf0, semx)
        for ch in range(n_ch):
            xpend.wait()
            xbuf = xbufs[ch % 2]
            if ch + 1 < n_ch:
                xpend = pltpu.async_copy(
                    x_hbm.at[pl.ds(base + (ch + 1) * _RS_CHUNK, _RS_CHUNK),
                             :], xbufs[(ch + 1) % 2], semx)
            for r in range(NREL):
                b = r % 2
                if pend[b] is not None:
                    pend[b].wait()
                ob, rv = obufs[b], relv[r, pl.ds(0, Q)]

                @plsc.parallel_loop(0, _RS_CHUNK, unroll=8)
                def _mul(i, ob=ob, rv=rv, xb=xbuf):
                    ob[i, pl.ds(0, Q)] = xb[i, pl.ds(0, Q)] * rv
                o0 = ((qid * NREL + r) * N + si * _RS_PER_W
                      + ch * _RS_CHUNK)
                pend[b] = pltpu.async_copy(
                    ob, out_hbm.at[pl.ds(o0, _RS_CHUNK), :], sems[b])
        for b in range(2):
            if pend[b] is not None:
                pend[b].wait()

    return rs_kernel


_rel_scale = _rel_scale_make()


def _mm_body(a0_ref, a1_ref, a2_ref, a3_ref, w_ref, out_ref):
    a = jnp.concatenate(
        [a0_ref[...], a1_ref[...], a2_ref[...], a3_ref[...]], axis=1)
    w = w_ref[...].reshape(DIM, Q)
    r = jnp.dot(a, w, preferred_element_type=jnp.float32)
    out_ref[...] = jnp.maximum(r, 0.0)


def _mm_relu(agg_q, w):
    grid = (4 * _NRB,)
    call = pl.pallas_call(
        _mm_body,
        grid=grid,
        in_specs=[
            pl.BlockSpec((_BLK, Q), lambda j: (0 * _NRB + j % _NRB, 0)),
            pl.BlockSpec((_BLK, Q), lambda j: (1 * _NRB + j % _NRB, 0)),
            pl.BlockSpec((_BLK, Q), lambda j: (2 * _NRB + j % _NRB, 0)),
            pl.BlockSpec((_BLK, Q), lambda j: (3 * _NRB + j % _NRB, 0)),
            pl.BlockSpec((1, DIM, Q), lambda j: (j // _NRB, 0, 0)),
        ],
        out_specs=pl.BlockSpec((_BLK, Q), lambda j: (j, 0)),
        out_shape=jax.ShapeDtypeStruct((4 * N, Q), jnp.float32),
    )
    w4 = jnp.stack([w[:, i * Q:(i + 1) * Q] for i in range(4)])
    return call(agg_q, agg_q, agg_q, agg_q, w4)


_IT_PER_W = B * L // 32
_IT_CHUNK = 400
_U_PER_W = B // 32
_P_PER_W = B * T // 32


def _gather_kernel_make():
    mesh = plsc.VectorSubcoreMesh(core_axis_name="c", subcore_axis_name="s")
    scratch = [
        pltpu.VMEM((_IT_PER_W,), jnp.int32),
        pltpu.VMEM((_IT_CHUNK, Q), jnp.float32),
        pltpu.VMEM((_P_PER_W, AD), jnp.float32),
        pltpu.VMEM((_P_PER_W,), jnp.float32),
        pltpu.SemaphoreType.DMA,
    ]
    out_type = [
        jax.ShapeDtypeStruct((B, AD), jnp.float32),
        jax.ShapeDtypeStruct((B * L, AD), jnp.float32),
        jax.ShapeDtypeStruct((B * T, AD), jnp.float32),
        jax.ShapeDtypeStruct((B * T,), jnp.float32),
    ]

    @functools.partial(pl.kernel, mesh=mesh, out_type=out_type,
                       scratch_types=scratch, compiler_params=_SC_PARAMS)
    def gather_kernel(x1, x2, x3, bu4, seq4, itp, pw, pb,
                      out_u, out_i, out_w, out_b,
                      idxv, rows, prow, pbv, sem):
        cid = lax.axis_index("c")
        sid = lax.axis_index("s")
        w = sid * 2 + cid

        ib = w * _IT_PER_W
        subs = ((0, 128), (128, 128), (256, 128), (384, 16))
        for q in range(4):
            pltpu.sync_copy(seq4.at[q, pl.ds(ib, _IT_PER_W)], idxv)
            for ch in range(4):
                for t, tb in enumerate((x1, x2, x3)):
                    for off, sz in subs:
                        pltpu.async_copy(
                            tb.at[idxv.at[pl.ds(ch * _IT_CHUNK + off, sz)]],
                            rows.at[pl.ds(off, sz), :], sem).wait()
                    pltpu.sync_copy(
                        rows,
                        out_i.at[pl.ds(ib + ch * _IT_CHUNK, _IT_CHUNK),
                                 pl.ds(t * DIM + q * Q, Q)])

        ub = w * _U_PER_W
        for q in range(4):
            pltpu.sync_copy(bu4.at[q, pl.ds(ub, _U_PER_W)],
                            idxv.at[pl.ds(0, _U_PER_W)])
            for t, tb in enumerate((x1, x2, x3)):
                pltpu.async_copy(tb.at[idxv.at[pl.ds(0, _U_PER_W)]],
                                 rows.at[pl.ds(0, _U_PER_W), :], sem).wait()
                pltpu.sync_copy(
                    rows.at[pl.ds(0, _U_PER_W), :],
                    out_u.at[pl.ds(ub, _U_PER_W),
                             pl.ds(t * DIM + q * Q, Q)])

        pbase = w * _P_PER_W
        pltpu.sync_copy(itp.at[pl.ds(pbase, _P_PER_W)],
                        idxv.at[pl.ds(0, _P_PER_W)])
        for off, sz in ((0, 128), (128, 64)):
            pltpu.async_copy(pw.at[idxv.at[pl.ds(off, sz)]],
                             prow.at[pl.ds(off, sz), :], sem).wait()
            pltpu.async_copy(pb.at[idxv.at[pl.ds(off, sz)]],
                             pbv.at[pl.ds(off, sz)], sem).wait()
        pltpu.sync_copy(prow, out_w.at[pl.ds(pbase, _P_PER_W), :])
        pltpu.sync_copy(pbv, out_b.at[pl.ds(pbase, _P_PER_W)])

    return gather_kernel


_gather_rows = _gather_kernel_make()

_NB = 64
_SCALE = 1.0 / math.sqrt(float(AD))


def _attn_body(user_ref, item_ref, pew_ref, peb_ref,
               wq_ref, wk_ref, wv_ref, wcq_ref, wck_ref, wcv_ref,
               res_ref, item_out_ref):
    item = item_ref[...]
    user = user_ref[...]
    q = jnp.dot(item, wq_ref[...], preferred_element_type=jnp.float32)
    k = jnp.dot(item, wk_ref[...], preferred_element_type=jnp.float32)
    v = jnp.dot(item, wv_ref[...], preferred_element_type=jnp.float32)
    q3 = q.reshape(_NB, L, AD)
    k3 = k.reshape(_NB, L, AD)
    v3 = v.reshape(_NB, L, AD)
    s = lax.dot_general(q3, k3, (((2,), (2,)), ((0,), (0,))),
                        preferred_element_type=jnp.float32) * _SCALE
    s = s - jnp.max(s, axis=-1, keepdims=True)
    e = jnp.exp(s)
    attn = e / jnp.sum(e, axis=-1, keepdims=True)
    it3 = item.reshape(_NB, L, AD) + lax.dot_general(
        attn, v3, (((2,), (1,)), ((0,), (0,))),
        preferred_element_type=jnp.float32)
    qc = jnp.dot(it3.reshape(_NB * L, AD), wcq_ref[...],
                 preferred_element_type=jnp.float32).reshape(_NB, L, AD)
    kc = jnp.dot(user, wck_ref[...], preferred_element_type=jnp.float32)
    vc = jnp.dot(user, wcv_ref[...], preferred_element_type=jnp.float32)
    score = jnp.sum(qc * kc[:, None, :], axis=-1) * _SCALE
    score = score - jnp.max(score, axis=-1, keepdims=True)
    es = jnp.exp(score)
    alpha = es / jnp.sum(es, axis=-1, keepdims=True)
    it4 = it3 + alpha[:, :, None] * vc[:, None, :]
    item_out_ref[...] = it4.reshape(_NB * L, AD)
    isum = jnp.sum(it4, axis=1)
    up = user + isum
    pew = pew_ref[...].reshape(_NB, T, AD)
    res_ref[...] = peb_ref[...] + jnp.sum(pew * up[:, None, :], axis=-1)


def _attention(user0, item0, pew, peb, wq, wk, wv, wcq, wck, wcv):
    grid = (B // _NB,)
    wspec = pl.BlockSpec((AD, AD), lambda i: (0, 0))
    return pl.pallas_call(
        _attn_body,
        grid=grid,
        in_specs=[
            pl.BlockSpec((_NB, AD), lambda i: (i, 0)),
            pl.BlockSpec((_NB * L, AD), lambda i: (i, 0)),
            pl.BlockSpec((_NB * T, AD), lambda i: (i, 0)),
            pl.BlockSpec((_NB, T), lambda i: (i, 0)),
            wspec, wspec, wspec, wspec, wspec, wspec,
        ],
        out_specs=[
            pl.BlockSpec((_NB, T), lambda i: (i, 0)),
            pl.BlockSpec((_NB * L, AD), lambda i: (i, 0)),
        ],
        out_shape=[
            jax.ShapeDtypeStruct((B, T), jnp.float32),
            jax.ShapeDtypeStruct((B * L, AD), jnp.float32),
        ],
    )(user0, item0, pew, peb, wq, wk, wv, wcq, wck, wcv)


def _pad_edges(gidx, dst, e_pad, sect_rows):
    e = gidx.shape[0]
    npad = e_pad - e
    padi = jnp.arange(npad, dtype=jnp.int32)
    g_p = jnp.concatenate([gidx, padi % N])
    dst_p = jnp.concatenate([dst, N + (padi % (ACC_ROWS - N))])
    g4 = jnp.stack([g_p + q * sect_rows for q in range(4)])
    return g4, dst_p


def _to_quarters(x):
    m = x.shape[0]
    return x.reshape(m, 4, Q).transpose(1, 0, 2).reshape(4 * m, Q)


def kernel(batch_users, batch_sequences, items_to_predict, edge_index,
           edge_type, node_no, short_edge_index, node_emb, rel_emb,
           W_long0, W_long1, W_short0, Wq, Wk, Wv, Wcq, Wck, Wcv,
           predict_w, predict_b):
    src, dst = edge_index[0], edge_index[1]
    s_src, s_dst = short_edge_index[0], short_edge_index[1]
    gidxL = edge_type * N + src
    g4L, d2L = _pad_edges(gidxL, dst, E_LONG_PAD, 8 * N)
    g4S, d2S = _pad_edges(s_src, s_dst, E_SHORT_PAD, N)

    xq0 = _to_quarters(node_emb)
    rel4 = _to_quarters(rel_emb)

    sc0 = _rel_scale(xq0, rel4)
    agg1 = _segsum_long(g4L, d2L, sc0)
    x1q = _mm_relu(agg1, W_long0)
    sc1 = _rel_scale(x1q, rel4)
    agg2 = _segsum_long(g4L, d2L, sc1)
    x2q = _mm_relu(agg2, W_long1)
    agg3 = _segsum_short(g4S, d2S, x2q)
    x3q = _mm_relu(agg3, W_short0)

    seq = batch_sequences.reshape(-1)
    seq4 = jnp.stack([seq + q * N for q in range(4)])
    bu4 = jnp.stack([batch_users + q * N for q in range(4)])
    user0, item0, pew, peb = _gather_rows(
        x1q, x2q, x3q, bu4, seq4,
        items_to_predict.reshape(-1), predict_w, predict_b[:, 0])

    res, item_out = _attention(user0, item0, pew, peb.reshape(B, T),
                               Wq, Wk, Wv, Wcq, Wck, Wcv)
    return (res, user0, item_out.reshape(B, L, AD))
